# Initial kernel scaffold; baseline (speedup 1.0000x reference)
import jax
import jax.numpy as jnp
from jax.experimental import pallas as pl


def kernel(data, org_edge_index, emb, lin_W, lin_b, att_i, att_j, gnn_bias, bn1_g, bn1_b, bn2_g, bn2_b, out_W, out_b):
    raise NotImplementedError("write your pallas kernel here")


# speedup vs baseline: 14.9163x; 14.9163x over previous
import functools

import jax
import jax.numpy as jnp
from jax import lax
from jax.experimental import pallas as pl
from jax.experimental.pallas import tpu as pltpu
from jax.experimental.pallas import tpu_sc as plsc

N = 10000
B = 4
FIN = 10
DIM = 64
TOPK = 16
M = B * N
NPAD = 10112
RB = 128
RP = 1000
NBLK = M // RP
EBLK = N // RP
EPS = 1e-5

NTILES = 32
CH = M // NTILES
CHP = 1256
GPN = 5
NG = CH // GPN
IDXW = GPN * TOPK


def _embn_body(e_ref, o_ref):
    x = e_ref[...]
    nrm = jnp.sqrt(jnp.sum(x * x, axis=1, keepdims=True))
    o_ref[...] = x / (nrm + 1e-12)


def _embn(embp):
    return pl.pallas_call(
        _embn_body,
        grid=(8,),
        in_specs=[pl.BlockSpec((NPAD // 8, DIM), lambda i: (i, 0))],
        out_specs=pl.BlockSpec((NPAD // 8, DIM), lambda i: (i, 0)),
        out_shape=jax.ShapeDtypeStruct((NPAD, DIM), jnp.float32),
    )(embp)


def _topk_body(a_ref, all_ref, idx_ref):
    a = a_ref[...]
    bfull = all_ref[...]
    sim = lax.dot_general(a, bfull, (((1,), (1,)), ((), ())),
                          preferred_element_type=jnp.float32)
    col = lax.broadcasted_iota(jnp.int32, (RB, NPAD), 1)
    neg = jnp.float32(-3e38)
    sim = jnp.where(col < N, sim, neg)
    big = jnp.int32(2**30)
    cols = []
    for _ in range(TOPK):
        m = jnp.max(sim, axis=1, keepdims=True)
        cand = jnp.where(sim >= m, col, big)
        idx = jnp.min(cand, axis=1, keepdims=True)
        cols.append(idx)
        sim = jnp.where(col == idx, neg, sim)
    idx_ref[...] = jnp.concatenate(cols, axis=1)


def _topk(embn):
    return pl.pallas_call(
        _topk_body,
        grid=(NPAD // RB,),
        in_specs=[pl.BlockSpec((RB, DIM), lambda i: (i, 0)),
                  pl.BlockSpec((NPAD, DIM), lambda i: (0, 0))],
        out_specs=pl.BlockSpec((RB, TOPK), lambda i: (i, 0)),
        out_shape=jax.ShapeDtypeStruct((NPAD, TOPK), jnp.int32),
        compiler_params=pltpu.CompilerParams(
            vmem_limit_bytes=100 * 1024 * 1024),
    )(embn, embn)


def _prep_body(x_ref, e_ref, w_ref, b_ref, aix_ref, aie_ref, ajx_ref, aje_ref,
               xp_ref, ai_ref, aj_ref):
    x = x_ref[...]
    emb = e_ref[...]
    xp = lax.dot_general(x, w_ref[...], (((1,), (0,)), ((), ())),
                         preferred_element_type=jnp.float32) + b_ref[...]
    xp_ref[...] = jnp.concatenate(
        [xp, jnp.zeros((RP, 128 - DIM), jnp.float32)], axis=1)
    ai = jnp.sum(xp * aix_ref[...], axis=1) + jnp.sum(emb * aie_ref[...], axis=1)
    aj = jnp.sum(xp * ajx_ref[...], axis=1) + jnp.sum(emb * aje_ref[...], axis=1)
    ai_ref[...] = ai.reshape(1, 1, RP)
    aj_ref[...] = aj.reshape(1, 1, RP)


def _prep(xflat, emb, lin_W, lin_b2, aix, aie, ajx, aje):
    return pl.pallas_call(
        _prep_body,
        grid=(NBLK,),
        in_specs=[pl.BlockSpec((RP, FIN), lambda i: (i, 0)),
                  pl.BlockSpec((RP, DIM), lambda i: (i % EBLK, 0)),
                  pl.BlockSpec((FIN, DIM), lambda i: (0, 0)),
                  pl.BlockSpec((1, DIM), lambda i: (0, 0)),
                  pl.BlockSpec((1, DIM), lambda i: (0, 0)),
                  pl.BlockSpec((1, DIM), lambda i: (0, 0)),
                  pl.BlockSpec((1, DIM), lambda i: (0, 0)),
                  pl.BlockSpec((1, DIM), lambda i: (0, 0))],
        out_specs=[pl.BlockSpec((RP, 128), lambda i: (i, 0)),
                   pl.BlockSpec((1, 1, RP), lambda i: (i, 0, 0)),
                   pl.BlockSpec((1, 1, RP), lambda i: (i, 0, 0))],
        out_shape=[jax.ShapeDtypeStruct((M, 128), jnp.float32),
                   jax.ShapeDtypeStruct((NBLK, 1, RP), jnp.float32),
                   jax.ShapeDtypeStruct((NBLK, 1, RP), jnp.float32)],
    )(xflat, emb, lin_W, lin_b2, aix, aie, ajx, aje)


def _msg_sc_body(idx_hbm, aj_hbm, ai_hbm, xp_hbm, out_hbm,
                 idx_v, aj_v, ai_v, rows_v, outb0_v, outb1_v,
                 sg0, sg1, so0, so1):
    c = lax.axis_index("c")
    s = lax.axis_index("s")
    wid = c * 16 + s
    node0 = wid * CH

    pltpu.sync_copy(idx_hbm.at[pl.ds(wid * (NG * IDXW), NG * IDXW)], idx_v)
    pltpu.sync_copy(aj_hbm, aj_v)
    pltpu.sync_copy(ai_hbm.at[pl.ds(wid * CHP, CHP)], ai_v)

    gsem = [sg0, sg1]
    osem = [so0, so1]
    outb = [outb0_v, outb1_v]

    def _gissue(g, p):
        pltpu.async_copy(xp_hbm.at[idx_v.at[pl.ds(g * IDXW, IDXW)]],
                         rows_v.at[p], gsem[p])

    def _gwait(g, p):
        pltpu.make_async_copy(xp_hbm.at[idx_v.at[pl.ds(g * IDXW, IDXW)]],
                              rows_v.at[p], gsem[p]).wait()

    def _oissue(g, p):
        pltpu.async_copy(
            outb[p],
            out_hbm.at[pl.ds((node0 + g * GPN) * DIM, GPN * DIM)], osem[p])

    def _owait(g, p):
        pltpu.make_async_copy(
            outb[p],
            out_hbm.at[pl.ds((node0 + g * GPN) * DIM, GPN * DIM)],
            osem[p]).wait()

    _gissue(0, 0)

    def _group(g, p):
        _gwait(g, p)

        @pl.when(g + 1 < NG)
        def _():
            _gissue(g + 1, 1 - p)

        @pl.when(g >= 2)
        def _():
            _owait(g - 2, p)

        for n in range(GPN):
            nid = g * GPN + n
            idxv = idx_v[pl.ds(g * IDXW + n * TOPK, TOPK)]
            lj = plsc.load_gather(aj_v, [idxv])
            aib = plsc.load_gather(ai_v, [jnp.full((TOPK,), nid, jnp.int32)])
            e = lj + aib
            e = jnp.where(e > 0, e, e * jnp.float32(0.2))
            mx = jnp.max(e)
            ex = jnp.exp(e - mx)
            den = jnp.sum(ex)
            alpha = ex / (den + jnp.float32(1e-16))
            dn = lax.GatherDimensionNumbers(
                offset_dims=(), collapsed_slice_dims=(0,),
                start_index_map=(0,))
            ak = [lax.gather(alpha, jnp.full((TOPK, 1), k, jnp.int32),
                             dn, (1,),
                             mode=lax.GatherScatterMode.PROMISE_IN_BOUNDS)
                  for k in range(TOPK)]
            for cg in range(DIM // 16):
                acc = ak[0] * rows_v[p, n * TOPK, pl.ds(cg * 16, 16)]
                for k in range(1, TOPK):
                    acc = acc + ak[k] * rows_v[p, n * TOPK + k,
                                               pl.ds(cg * 16, 16)]
                outb[p][pl.ds(n * DIM + cg * 16, 16)] = acc
        _oissue(g, p)

    def _pair(i2, carry):
        _group(i2 * 2, 0)
        _group(i2 * 2 + 1, 1)
        return carry

    lax.fori_loop(0, NG // 2, _pair, 0)
    _owait(NG - 2, 0)
    _owait(NG - 1, 1)


def _msg_sc(idx_t, aj_flat, ai_pad, xp):
    mesh = plsc.VectorSubcoreMesh(core_axis_name="c", subcore_axis_name="s")
    f = functools.partial(
        pl.kernel, _msg_sc_body, mesh=mesh,
        out_type=jax.ShapeDtypeStruct((M * DIM,), jnp.float32),
        scratch_types=[
            pltpu.VMEM((NG * IDXW,), jnp.int32),
            pltpu.VMEM((M,), jnp.float32),
            pltpu.VMEM((CHP,), jnp.float32),
            pltpu.VMEM((2, IDXW, 128), jnp.float32),
            pltpu.VMEM((GPN * DIM,), jnp.float32),
            pltpu.VMEM((GPN * DIM,), jnp.float32),
            pltpu.SemaphoreType.DMA,
            pltpu.SemaphoreType.DMA,
            pltpu.SemaphoreType.DMA,
            pltpu.SemaphoreType.DMA,
        ],
        compiler_params=pltpu.CompilerParams(needs_layout_passes=False),
    )()
    return f(idx_t, aj_flat, ai_pad, xp)


def _stats_body(x_ref, s_ref):
    @pl.when(pl.program_id(0) == 0)
    def _():
        s_ref[...] = jnp.zeros((8, DIM), jnp.float32)

    x = x_ref[...]
    s0 = jnp.sum(x, axis=0, keepdims=True)
    s1 = jnp.sum(x * x, axis=0, keepdims=True)
    upd = jnp.concatenate([s0, s1, jnp.zeros((6, DIM), jnp.float32)], axis=0)
    s_ref[...] = s_ref[...] + upd


def _stats(xflat):
    return pl.pallas_call(
        _stats_body,
        grid=(NBLK,),
        in_specs=[pl.BlockSpec((RP, DIM), lambda i: (i, 0))],
        out_specs=pl.BlockSpec((8, DIM), lambda i: (0, 0)),
        out_shape=jax.ShapeDtypeStruct((8, DIM), jnp.float32),
    )(xflat)


def _bn1_body(x_ref, e_ref, s_ref, g_ref, b_ref, o_ref, s2_ref):
    s = s_ref[...]
    mu = s[0:1, :] * (1.0 / M)
    var = s[1:2, :] * (1.0 / M) - mu * mu
    scale = g_ref[...] * lax.rsqrt(var + EPS)
    shift = b_ref[...] - mu * scale
    o = jnp.maximum(x_ref[...] * scale + shift, 0.0) * e_ref[...]
    o_ref[...] = o

    @pl.when(pl.program_id(0) == 0)
    def _():
        s2_ref[...] = jnp.zeros((8, DIM), jnp.float32)

    s0 = jnp.sum(o, axis=0, keepdims=True)
    s1 = jnp.sum(o * o, axis=0, keepdims=True)
    upd = jnp.concatenate([s0, s1, jnp.zeros((6, DIM), jnp.float32)], axis=0)
    s2_ref[...] = s2_ref[...] + upd


def _bn1(msg, emb, stats1, g1, b1):
    return pl.pallas_call(
        _bn1_body,
        grid=(NBLK,),
        in_specs=[pl.BlockSpec((RP, DIM), lambda i: (i, 0)),
                  pl.BlockSpec((RP, DIM), lambda i: (i % EBLK, 0)),
                  pl.BlockSpec((8, DIM), lambda i: (0, 0)),
                  pl.BlockSpec((1, DIM), lambda i: (0, 0)),
                  pl.BlockSpec((1, DIM), lambda i: (0, 0))],
        out_specs=[pl.BlockSpec((RP, DIM), lambda i: (i, 0)),
                   pl.BlockSpec((8, DIM), lambda i: (0, 0))],
        out_shape=[jax.ShapeDtypeStruct((M, DIM), jnp.float32),
                   jax.ShapeDtypeStruct((8, DIM), jnp.float32)],
    )(msg, emb, stats1, g1, b1)


def _out_body(o_ref, s_ref, g_ref, b_ref, w_ref, ob_ref, y_ref):
    s = s_ref[...]
    mu = s[0:1, :] * (1.0 / M)
    var = s[1:2, :] * (1.0 / M) - mu * mu
    scale = g_ref[...] * lax.rsqrt(var + EPS)
    shift = b_ref[...] - mu * scale
    o = jnp.maximum(o_ref[...] * scale + shift, 0.0)
    y = jnp.sum(o * w_ref[...], axis=1) + ob_ref[0, 0]
    y_ref[...] = y.reshape(1, 1, RP)


def _outk(o, stats2, g2, b2, wrow, ob):
    return pl.pallas_call(
        _out_body,
        grid=(NBLK,),
        in_specs=[pl.BlockSpec((RP, DIM), lambda i: (i, 0)),
                  pl.BlockSpec((8, DIM), lambda i: (0, 0)),
                  pl.BlockSpec((1, DIM), lambda i: (0, 0)),
                  pl.BlockSpec((1, DIM), lambda i: (0, 0)),
                  pl.BlockSpec((1, DIM), lambda i: (0, 0)),
                  pl.BlockSpec((1, 1), lambda i: (0, 0))],
        out_specs=pl.BlockSpec((1, 1, RP), lambda i: (i, 0, 0)),
        out_shape=jax.ShapeDtypeStruct((NBLK, 1, RP), jnp.float32),
    )(o, stats2, g2, b2, wrow, ob)


def kernel(data, org_edge_index, emb, lin_W, lin_b, att_i, att_j, gnn_bias,
           bn1_g, bn1_b, bn2_g, bn2_b, out_W, out_b):
    del org_edge_index, gnn_bias

    embp = jnp.pad(emb, ((0, NPAD - N), (0, 0)))
    embn = _embn(embp)
    topk_idx = _topk(embn)[:N]

    xflat = data.reshape(M, FIN)
    xp, ai3, aj3 = _prep(
        xflat, emb, lin_W, lin_b.reshape(1, DIM),
        att_i[:DIM].reshape(1, DIM), att_i[DIM:].reshape(1, DIM),
        att_j[:DIM].reshape(1, DIM), att_j[DIM:].reshape(1, DIM))
    ai = ai3.reshape(M)
    aj = aj3.reshape(M)

    offs = (jnp.arange(B, dtype=jnp.int32) * N)[:, None, None]
    gidx = (topk_idx[None].astype(jnp.int32) + offs).reshape(-1)
    ai_pad = jnp.pad(ai.reshape(NTILES, CH),
                     ((0, 0), (0, CHP - CH))).reshape(-1)

    msg = _msg_sc(gidx, aj, ai_pad, xp).reshape(M, DIM)

    stats1 = _stats(msg)
    o, stats2 = _bn1(msg, emb, stats1, bn1_g.reshape(1, DIM),
                     bn1_b.reshape(1, DIM))
    y = _outk(o, stats2, bn2_g.reshape(1, DIM), bn2_b.reshape(1, DIM),
              out_W.reshape(1, DIM), out_b.reshape(1, 1))
    return y.reshape(B, N)


# speedup vs baseline: 26.8467x; 1.7998x over previous
import functools

import jax
import jax.numpy as jnp
from jax import lax
from jax.experimental import pallas as pl
from jax.experimental.pallas import tpu as pltpu
from jax.experimental.pallas import tpu_sc as plsc

N = 10000
B = 4
FIN = 10
DIM = 64
TOPK = 16
M = B * N
NPAD = 10112
RB = 128
RP = 1000
NBLK = M // RP
EBLK = N // RP
EPS = 1e-5

NTILES = 32
CH = M // NTILES
CHP = 1256
GPN = 5
NG = CH // GPN
IDXW = GPN * TOPK


def _embn_body(e_ref, o_ref):
    x = e_ref[...]
    nrm = jnp.sqrt(jnp.sum(x * x, axis=1, keepdims=True))
    o_ref[...] = x / (nrm + 1e-12)


def _embn(embp):
    return pl.pallas_call(
        _embn_body,
        grid=(8,),
        in_specs=[pl.BlockSpec((NPAD // 8, DIM), lambda i: (i, 0))],
        out_specs=pl.BlockSpec((NPAD // 8, DIM), lambda i: (i, 0)),
        out_shape=jax.ShapeDtypeStruct((NPAD, DIM), jnp.float32),
    )(embp)


NCH = NPAD // 128


def _topk_body(a_ref, all_ref, idx_ref):
    a = a_ref[...]
    bfull = all_ref[...]
    sim = lax.dot_general(a, bfull, (((1,), (1,)), ((), ())),
                          preferred_element_type=jnp.float32)
    neg = jnp.float32(-3e38)
    big = jnp.int32(2**30)
    lanes = lax.broadcasted_iota(jnp.int32, (RB, 128), 1)
    zed = jnp.zeros((RB, 128), jnp.int32)

    def chunk(c):
        s = sim[:, c * 128:(c + 1) * 128]
        if (c + 1) * 128 > N:
            s = jnp.where(lanes < N - c * 128, s, neg)
        return s
    v0 = chunk(0)
    c0 = zed
    v1 = v2 = v3 = jnp.full((RB, 128), neg)
    c1 = c2 = c3 = zed
    for c in range(1, NCH):
        s = chunk(c)
        b0 = s > v0
        b1 = s > v1
        b2 = s > v2
        b3 = s > v3
        v3 = jnp.where(b3, jnp.where(b2, v2, s), v3)
        c3 = jnp.where(b3, jnp.where(b2, c2, c), c3)
        v2 = jnp.where(b2, jnp.where(b1, v1, s), v2)
        c2 = jnp.where(b2, jnp.where(b1, c1, c), c2)
        v1 = jnp.where(b1, jnp.where(b0, v0, s), v1)
        c1 = jnp.where(b1, jnp.where(b0, c0, c), c1)
        v0 = jnp.where(b0, s, v0)
        c0 = jnp.where(b0, c, c0)

    V = jnp.concatenate([v0, v1, v2, v3], axis=1)
    G = jnp.concatenate([c0 * 128 + lanes, c1 * 128 + lanes,
                         c2 * 128 + lanes, c3 * 128 + lanes], axis=1)
    cols = []
    v16 = None
    for k in range(TOPK):
        m = jnp.max(V, axis=1, keepdims=True)
        candi = jnp.where(V >= m, G, big)
        mi = jnp.min(candi, axis=1, keepdims=True)
        cols.append(mi)
        V = jnp.where(candi == mi, neg, V)
        if k == TOPK - 1:
            v16 = m
    idx_ref[...] = jnp.concatenate(cols, axis=1)

    colf = lax.broadcasted_iota(jnp.int32, (RB, NPAD), 1)
    valid = colf < N
    gt = jnp.sum(jnp.where(valid & (sim > v16), 1.0, 0.0), axis=1,
                 keepdims=True)
    ge = jnp.sum(jnp.where(valid & (sim >= v16), 1.0, 0.0), axis=1,
                 keepdims=True)
    nbad = jnp.sum(jnp.where((gt <= 15.0) & (ge == 16.0), 0.0, 1.0))

    @pl.when(nbad > 0.0)
    def _():
        smat = jnp.where(valid, sim, neg)
        outc = []
        for _ in range(TOPK):
            m = jnp.max(smat, axis=1, keepdims=True)
            cand = jnp.where(smat >= m, colf, big)
            idx = jnp.min(cand, axis=1, keepdims=True)
            outc.append(idx)
            smat = jnp.where(colf == idx, neg, smat)
        idx_ref[...] = jnp.concatenate(outc, axis=1)


def _topk(embn):
    return pl.pallas_call(
        _topk_body,
        grid=(NPAD // RB,),
        in_specs=[pl.BlockSpec((RB, DIM), lambda i: (i, 0)),
                  pl.BlockSpec((NPAD, DIM), lambda i: (0, 0))],
        out_specs=pl.BlockSpec((RB, TOPK), lambda i: (i, 0)),
        out_shape=jax.ShapeDtypeStruct((NPAD, TOPK), jnp.int32),
        compiler_params=pltpu.CompilerParams(
            vmem_limit_bytes=100 * 1024 * 1024),
    )(embn, embn)


def _prep_body(x_ref, e_ref, w_ref, b_ref, aix_ref, aie_ref, ajx_ref, aje_ref,
               xp_ref, ai_ref, aj_ref):
    x = x_ref[...]
    emb = e_ref[...]
    xp = lax.dot_general(x, w_ref[...], (((1,), (0,)), ((), ())),
                         preferred_element_type=jnp.float32) + b_ref[...]
    xp_ref[...] = jnp.concatenate(
        [xp, jnp.zeros((RP, 128 - DIM), jnp.float32)], axis=1)
    ai = jnp.sum(xp * aix_ref[...], axis=1) + jnp.sum(emb * aie_ref[...], axis=1)
    aj = jnp.sum(xp * ajx_ref[...], axis=1) + jnp.sum(emb * aje_ref[...], axis=1)
    ai_ref[...] = ai.reshape(1, 1, RP)
    aj_ref[...] = aj.reshape(1, 1, RP)


def _prep(xflat, emb, lin_W, lin_b2, aix, aie, ajx, aje):
    return pl.pallas_call(
        _prep_body,
        grid=(NBLK,),
        in_specs=[pl.BlockSpec((RP, FIN), lambda i: (i, 0)),
                  pl.BlockSpec((RP, DIM), lambda i: (i % EBLK, 0)),
                  pl.BlockSpec((FIN, DIM), lambda i: (0, 0)),
                  pl.BlockSpec((1, DIM), lambda i: (0, 0)),
                  pl.BlockSpec((1, DIM), lambda i: (0, 0)),
                  pl.BlockSpec((1, DIM), lambda i: (0, 0)),
                  pl.BlockSpec((1, DIM), lambda i: (0, 0)),
                  pl.BlockSpec((1, DIM), lambda i: (0, 0))],
        out_specs=[pl.BlockSpec((RP, 128), lambda i: (i, 0)),
                   pl.BlockSpec((1, 1, RP), lambda i: (i, 0, 0)),
                   pl.BlockSpec((1, 1, RP), lambda i: (i, 0, 0))],
        out_shape=[jax.ShapeDtypeStruct((M, 128), jnp.float32),
                   jax.ShapeDtypeStruct((NBLK, 1, RP), jnp.float32),
                   jax.ShapeDtypeStruct((NBLK, 1, RP), jnp.float32)],
    )(xflat, emb, lin_W, lin_b2, aix, aie, ajx, aje)


def _msg_sc_body(idx_hbm, aj_hbm, ai_hbm, xp_hbm, out_hbm,
                 idx_v, aj_v, ai_v, rows_v, outb0_v, outb1_v,
                 sg0, sg1, so0, so1):
    c = lax.axis_index("c")
    s = lax.axis_index("s")
    wid = c * 16 + s
    node0 = wid * CH

    pltpu.sync_copy(idx_hbm.at[pl.ds(wid * (NG * IDXW), NG * IDXW)], idx_v)
    pltpu.sync_copy(aj_hbm, aj_v)
    pltpu.sync_copy(ai_hbm.at[pl.ds(wid * CHP, CHP)], ai_v)

    gsem = [sg0, sg1]
    osem = [so0, so1]
    outb = [outb0_v, outb1_v]

    def _gissue(g, p):
        pltpu.async_copy(xp_hbm.at[idx_v.at[pl.ds(g * IDXW, IDXW)]],
                         rows_v.at[p], gsem[p])

    def _gwait(g, p):
        pltpu.make_async_copy(xp_hbm.at[idx_v.at[pl.ds(g * IDXW, IDXW)]],
                              rows_v.at[p], gsem[p]).wait()

    def _oissue(g, p):
        pltpu.async_copy(
            outb[p],
            out_hbm.at[pl.ds((node0 + g * GPN) * DIM, GPN * DIM)], osem[p])

    def _owait(g, p):
        pltpu.make_async_copy(
            outb[p],
            out_hbm.at[pl.ds((node0 + g * GPN) * DIM, GPN * DIM)],
            osem[p]).wait()

    _gissue(0, 0)

    def _group(g, p):
        _gwait(g, p)

        @pl.when(g + 1 < NG)
        def _():
            _gissue(g + 1, 1 - p)

        @pl.when(g >= 2)
        def _():
            _owait(g - 2, p)

        for n in range(GPN):
            nid = g * GPN + n
            idxv = idx_v[pl.ds(g * IDXW + n * TOPK, TOPK)]
            lj = plsc.load_gather(aj_v, [idxv])
            aib = plsc.load_gather(ai_v, [jnp.full((TOPK,), nid, jnp.int32)])
            e = lj + aib
            e = jnp.where(e > 0, e, e * jnp.float32(0.2))
            mx = jnp.max(e)
            ex = jnp.exp(e - mx)
            den = jnp.sum(ex)
            alpha = ex / (den + jnp.float32(1e-16))
            dn = lax.GatherDimensionNumbers(
                offset_dims=(), collapsed_slice_dims=(0,),
                start_index_map=(0,))
            ak = [lax.gather(alpha, jnp.full((TOPK, 1), k, jnp.int32),
                             dn, (1,),
                             mode=lax.GatherScatterMode.PROMISE_IN_BOUNDS)
                  for k in range(TOPK)]
            for cg in range(DIM // 16):
                acc = ak[0] * rows_v[p, n * TOPK, pl.ds(cg * 16, 16)]
                for k in range(1, TOPK):
                    acc = acc + ak[k] * rows_v[p, n * TOPK + k,
                                               pl.ds(cg * 16, 16)]
                outb[p][pl.ds(n * DIM + cg * 16, 16)] = acc
        _oissue(g, p)

    def _pair(i2, carry):
        _group(i2 * 2, 0)
        _group(i2 * 2 + 1, 1)
        return carry

    lax.fori_loop(0, NG // 2, _pair, 0)
    _owait(NG - 2, 0)
    _owait(NG - 1, 1)


def _msg_sc(idx_t, aj_flat, ai_pad, xp):
    mesh = plsc.VectorSubcoreMesh(core_axis_name="c", subcore_axis_name="s")
    f = functools.partial(
        pl.kernel, _msg_sc_body, mesh=mesh,
        out_type=jax.ShapeDtypeStruct((M * DIM,), jnp.float32),
        scratch_types=[
            pltpu.VMEM((NG * IDXW,), jnp.int32),
            pltpu.VMEM((M,), jnp.float32),
            pltpu.VMEM((CHP,), jnp.float32),
            pltpu.VMEM((2, IDXW, 128), jnp.float32),
            pltpu.VMEM((GPN * DIM,), jnp.float32),
            pltpu.VMEM((GPN * DIM,), jnp.float32),
            pltpu.SemaphoreType.DMA,
            pltpu.SemaphoreType.DMA,
            pltpu.SemaphoreType.DMA,
            pltpu.SemaphoreType.DMA,
        ],
        compiler_params=pltpu.CompilerParams(needs_layout_passes=False),
    )()
    return f(idx_t, aj_flat, ai_pad, xp)


def _stats_body(x_ref, s_ref):
    @pl.when(pl.program_id(0) == 0)
    def _():
        s_ref[...] = jnp.zeros((8, DIM), jnp.float32)

    x = x_ref[...]
    s0 = jnp.sum(x, axis=0, keepdims=True)
    s1 = jnp.sum(x * x, axis=0, keepdims=True)
    upd = jnp.concatenate([s0, s1, jnp.zeros((6, DIM), jnp.float32)], axis=0)
    s_ref[...] = s_ref[...] + upd


def _stats(xflat):
    return pl.pallas_call(
        _stats_body,
        grid=(NBLK,),
        in_specs=[pl.BlockSpec((RP, DIM), lambda i: (i, 0))],
        out_specs=pl.BlockSpec((8, DIM), lambda i: (0, 0)),
        out_shape=jax.ShapeDtypeStruct((8, DIM), jnp.float32),
    )(xflat)


def _bn1_body(x_ref, e_ref, s_ref, g_ref, b_ref, o_ref, s2_ref):
    s = s_ref[...]
    mu = s[0:1, :] * (1.0 / M)
    var = s[1:2, :] * (1.0 / M) - mu * mu
    scale = g_ref[...] * lax.rsqrt(var + EPS)
    shift = b_ref[...] - mu * scale
    o = jnp.maximum(x_ref[...] * scale + shift, 0.0) * e_ref[...]
    o_ref[...] = o

    @pl.when(pl.program_id(0) == 0)
    def _():
        s2_ref[...] = jnp.zeros((8, DIM), jnp.float32)

    s0 = jnp.sum(o, axis=0, keepdims=True)
    s1 = jnp.sum(o * o, axis=0, keepdims=True)
    upd = jnp.concatenate([s0, s1, jnp.zeros((6, DIM), jnp.float32)], axis=0)
    s2_ref[...] = s2_ref[...] + upd


def _bn1(msg, emb, stats1, g1, b1):
    return pl.pallas_call(
        _bn1_body,
        grid=(NBLK,),
        in_specs=[pl.BlockSpec((RP, DIM), lambda i: (i, 0)),
                  pl.BlockSpec((RP, DIM), lambda i: (i % EBLK, 0)),
                  pl.BlockSpec((8, DIM), lambda i: (0, 0)),
                  pl.BlockSpec((1, DIM), lambda i: (0, 0)),
                  pl.BlockSpec((1, DIM), lambda i: (0, 0))],
        out_specs=[pl.BlockSpec((RP, DIM), lambda i: (i, 0)),
                   pl.BlockSpec((8, DIM), lambda i: (0, 0))],
        out_shape=[jax.ShapeDtypeStruct((M, DIM), jnp.float32),
                   jax.ShapeDtypeStruct((8, DIM), jnp.float32)],
    )(msg, emb, stats1, g1, b1)


def _out_body(o_ref, s_ref, g_ref, b_ref, w_ref, ob_ref, y_ref):
    s = s_ref[...]
    mu = s[0:1, :] * (1.0 / M)
    var = s[1:2, :] * (1.0 / M) - mu * mu
    scale = g_ref[...] * lax.rsqrt(var + EPS)
    shift = b_ref[...] - mu * scale
    o = jnp.maximum(o_ref[...] * scale + shift, 0.0)
    y = jnp.sum(o * w_ref[...], axis=1) + ob_ref[0, 0]
    y_ref[...] = y.reshape(1, 1, RP)


def _outk(o, stats2, g2, b2, wrow, ob):
    return pl.pallas_call(
        _out_body,
        grid=(NBLK,),
        in_specs=[pl.BlockSpec((RP, DIM), lambda i: (i, 0)),
                  pl.BlockSpec((8, DIM), lambda i: (0, 0)),
                  pl.BlockSpec((1, DIM), lambda i: (0, 0)),
                  pl.BlockSpec((1, DIM), lambda i: (0, 0)),
                  pl.BlockSpec((1, DIM), lambda i: (0, 0)),
                  pl.BlockSpec((1, 1), lambda i: (0, 0))],
        out_specs=pl.BlockSpec((1, 1, RP), lambda i: (i, 0, 0)),
        out_shape=jax.ShapeDtypeStruct((NBLK, 1, RP), jnp.float32),
    )(o, stats2, g2, b2, wrow, ob)


def kernel(data, org_edge_index, emb, lin_W, lin_b, att_i, att_j, gnn_bias,
           bn1_g, bn1_b, bn2_g, bn2_b, out_W, out_b):
    del org_edge_index, gnn_bias

    embp = jnp.pad(emb, ((0, NPAD - N), (0, 0)))
    embn = _embn(embp)
    topk_idx = _topk(embn)[:N]

    xflat = data.reshape(M, FIN)
    xp, ai3, aj3 = _prep(
        xflat, emb, lin_W, lin_b.reshape(1, DIM),
        att_i[:DIM].reshape(1, DIM), att_i[DIM:].reshape(1, DIM),
        att_j[:DIM].reshape(1, DIM), att_j[DIM:].reshape(1, DIM))
    ai = ai3.reshape(M)
    aj = aj3.reshape(M)

    offs = (jnp.arange(B, dtype=jnp.int32) * N)[:, None, None]
    gidx = (topk_idx[None].astype(jnp.int32) + offs).reshape(-1)
    ai_pad = jnp.pad(ai.reshape(NTILES, CH),
                     ((0, 0), (0, CHP - CH))).reshape(-1)

    msg = _msg_sc(gidx, aj, ai_pad, xp).reshape(M, DIM)

    stats1 = _stats(msg)
    o, stats2 = _bn1(msg, emb, stats1, bn1_g.reshape(1, DIM),
                     bn1_b.reshape(1, DIM))
    y = _outk(o, stats2, bn2_g.reshape(1, DIM), bn2_b.reshape(1, DIM),
              out_W.reshape(1, DIM), out_b.reshape(1, 1))
    return y.reshape(B, N)


# speedup vs baseline: 27.6548x; 1.0301x over previous
import functools

import jax
import jax.numpy as jnp
from jax import lax
from jax.experimental import pallas as pl
from jax.experimental.pallas import tpu as pltpu
from jax.experimental.pallas import tpu_sc as plsc

N = 10000
B = 4
FIN = 10
DIM = 64
TOPK = 16
M = B * N
NPAD = 10112
RB = 128
RP = 1000
NBLK = M // RP
EBLK = N // RP
EPS = 1e-5

NTILES = 32
CH = M // NTILES
CHP = 1256
GPN = 5
NG = CH // GPN
IDXW = GPN * TOPK


def _embn_body(e_ref, o_ref):
    x = e_ref[...]
    nrm = jnp.sqrt(jnp.sum(x * x, axis=1, keepdims=True))
    o_ref[...] = x / (nrm + 1e-12)


def _embn(embp):
    return pl.pallas_call(
        _embn_body,
        grid=(8,),
        in_specs=[pl.BlockSpec((NPAD // 8, DIM), lambda i: (i, 0))],
        out_specs=pl.BlockSpec((NPAD // 8, DIM), lambda i: (i, 0)),
        out_shape=jax.ShapeDtypeStruct((NPAD, DIM), jnp.float32),
    )(embp)


NCH = NPAD // 128


def _topk_body(a_ref, all_ref, idx_ref):
    a = a_ref[...]
    bfull = all_ref[...]
    sim = lax.dot_general(a, bfull, (((1,), (1,)), ((), ())),
                          preferred_element_type=jnp.float32)
    neg = jnp.float32(-3e38)
    big = jnp.int32(2**30)
    lanes = lax.broadcasted_iota(jnp.int32, (RB, 128), 1)
    zed = jnp.zeros((RB, 128), jnp.int32)

    def chunk(c):
        s = sim[:, c * 128:(c + 1) * 128]
        if (c + 1) * 128 > N:
            s = jnp.where(lanes < N - c * 128, s, neg)
        return s
    v0 = chunk(0)
    c0 = zed
    v1 = v2 = v3 = jnp.full((RB, 128), neg)
    c1 = c2 = c3 = zed
    for c in range(1, NCH):
        s = chunk(c)
        b0 = s > v0
        b1 = s > v1
        b2 = s > v2
        b3 = s > v3
        v3 = jnp.where(b3, jnp.where(b2, v2, s), v3)
        c3 = jnp.where(b3, jnp.where(b2, c2, c), c3)
        v2 = jnp.where(b2, jnp.where(b1, v1, s), v2)
        c2 = jnp.where(b2, jnp.where(b1, c1, c), c2)
        v1 = jnp.where(b1, jnp.where(b0, v0, s), v1)
        c1 = jnp.where(b1, jnp.where(b0, c0, c), c1)
        v0 = jnp.where(b0, s, v0)
        c0 = jnp.where(b0, c, c0)

    V = jnp.concatenate([v0, v1, v2, v3], axis=1)
    G = jnp.concatenate([c0 * 128 + lanes, c1 * 128 + lanes,
                         c2 * 128 + lanes, c3 * 128 + lanes], axis=1)
    cols = []
    v16 = None
    for k in range(TOPK):
        m = jnp.max(V, axis=1, keepdims=True)
        candi = jnp.where(V >= m, G, big)
        mi = jnp.min(candi, axis=1, keepdims=True)
        cols.append(mi)
        V = jnp.where(candi == mi, neg, V)
        if k == TOPK - 1:
            v16 = m
    idx_ref[...] = jnp.concatenate(cols, axis=1)

    colf = lax.broadcasted_iota(jnp.int32, (RB, NPAD), 1)
    valid = colf < N
    ge = jnp.sum(jnp.where(valid & (sim >= v16), 1.0, 0.0), axis=1,
                 keepdims=True)
    nbad = jnp.sum(jnp.where(ge == 16.0, 0.0, 1.0))

    @pl.when(nbad > 0.0)
    def _():
        smat = jnp.where(valid, sim, neg)
        outc = []
        for _ in range(TOPK):
            m = jnp.max(smat, axis=1, keepdims=True)
            cand = jnp.where(smat >= m, colf, big)
            idx = jnp.min(cand, axis=1, keepdims=True)
            outc.append(idx)
            smat = jnp.where(colf == idx, neg, smat)
        idx_ref[...] = jnp.concatenate(outc, axis=1)


def _topk(embn):
    return pl.pallas_call(
        _topk_body,
        grid=(NPAD // RB,),
        in_specs=[pl.BlockSpec((RB, DIM), lambda i: (i, 0)),
                  pl.BlockSpec((NPAD, DIM), lambda i: (0, 0))],
        out_specs=pl.BlockSpec((RB, TOPK), lambda i: (i, 0)),
        out_shape=jax.ShapeDtypeStruct((NPAD, TOPK), jnp.int32),
        compiler_params=pltpu.CompilerParams(
            vmem_limit_bytes=100 * 1024 * 1024),
    )(embn, embn)


def _prep_body(x_ref, e_ref, w_ref, b_ref, aix_ref, aie_ref, ajx_ref, aje_ref,
               xp_ref, ai_ref, aj_ref):
    x = x_ref[...]
    emb = e_ref[...]
    xp = lax.dot_general(x, w_ref[...], (((1,), (0,)), ((), ())),
                         preferred_element_type=jnp.float32) + b_ref[...]
    xp_ref[...] = jnp.concatenate(
        [xp, jnp.zeros((RP, 128 - DIM), jnp.float32)], axis=1)
    ai = jnp.sum(xp * aix_ref[...], axis=1) + jnp.sum(emb * aie_ref[...], axis=1)
    aj = jnp.sum(xp * ajx_ref[...], axis=1) + jnp.sum(emb * aje_ref[...], axis=1)
    ai_ref[...] = ai.reshape(1, 1, RP)
    aj_ref[...] = aj.reshape(1, 1, RP)


def _prep(xflat, emb, lin_W, lin_b2, aix, aie, ajx, aje):
    return pl.pallas_call(
        _prep_body,
        grid=(NBLK,),
        in_specs=[pl.BlockSpec((RP, FIN), lambda i: (i, 0)),
                  pl.BlockSpec((RP, DIM), lambda i: (i % EBLK, 0)),
                  pl.BlockSpec((FIN, DIM), lambda i: (0, 0)),
                  pl.BlockSpec((1, DIM), lambda i: (0, 0)),
                  pl.BlockSpec((1, DIM), lambda i: (0, 0)),
                  pl.BlockSpec((1, DIM), lambda i: (0, 0)),
                  pl.BlockSpec((1, DIM), lambda i: (0, 0)),
                  pl.BlockSpec((1, DIM), lambda i: (0, 0))],
        out_specs=[pl.BlockSpec((RP, 128), lambda i: (i, 0)),
                   pl.BlockSpec((1, 1, RP), lambda i: (i, 0, 0)),
                   pl.BlockSpec((1, 1, RP), lambda i: (i, 0, 0))],
        out_shape=[jax.ShapeDtypeStruct((M, 128), jnp.float32),
                   jax.ShapeDtypeStruct((NBLK, 1, RP), jnp.float32),
                   jax.ShapeDtypeStruct((NBLK, 1, RP), jnp.float32)],
    )(xflat, emb, lin_W, lin_b2, aix, aie, ajx, aje)


def _msg_sc_body(idx_hbm, aj_hbm, ai_hbm, xp_hbm, out_hbm,
                 idx_v, aj_v, ai_v, rows_v, outb0_v, outb1_v,
                 sg0, sg1, so0, so1):
    c = lax.axis_index("c")
    s = lax.axis_index("s")
    wid = c * 16 + s
    node0 = wid * CH

    pltpu.sync_copy(idx_hbm.at[pl.ds(wid * (NG * IDXW), NG * IDXW)], idx_v)
    pltpu.sync_copy(aj_hbm, aj_v)
    pltpu.sync_copy(ai_hbm.at[pl.ds(wid * CHP, CHP)], ai_v)

    gsem = [sg0, sg1]
    osem = [so0, so1]
    outb = [outb0_v, outb1_v]

    def _gissue(g, p):
        pltpu.async_copy(xp_hbm.at[idx_v.at[pl.ds(g * IDXW, IDXW)]],
                         rows_v.at[p], gsem[p])

    def _gwait(g, p):
        pltpu.make_async_copy(xp_hbm.at[idx_v.at[pl.ds(g * IDXW, IDXW)]],
                              rows_v.at[p], gsem[p]).wait()

    def _oissue(g, p):
        pltpu.async_copy(
            outb[p],
            out_hbm.at[pl.ds((node0 + g * GPN) * DIM, GPN * DIM)], osem[p])

    def _owait(g, p):
        pltpu.make_async_copy(
            outb[p],
            out_hbm.at[pl.ds((node0 + g * GPN) * DIM, GPN * DIM)],
            osem[p]).wait()

    _gissue(0, 0)

    def _group(g, p):
        _gwait(g, p)

        @pl.when(g + 1 < NG)
        def _():
            _gissue(g + 1, 1 - p)

        @pl.when(g >= 2)
        def _():
            _owait(g - 2, p)

        for n in range(GPN):
            nid = g * GPN + n
            idxv = idx_v[pl.ds(g * IDXW + n * TOPK, TOPK)]
            lj = plsc.load_gather(aj_v, [idxv])
            aib = plsc.load_gather(ai_v, [jnp.full((TOPK,), nid, jnp.int32)])
            e = lj + aib
            e = jnp.where(e > 0, e, e * jnp.float32(0.2))
            mx = jnp.max(e)
            ex = jnp.exp(e - mx)
            den = jnp.sum(ex)
            alpha = ex / (den + jnp.float32(1e-16))
            dn = lax.GatherDimensionNumbers(
                offset_dims=(), collapsed_slice_dims=(0,),
                start_index_map=(0,))
            ak = [lax.gather(alpha, jnp.full((TOPK, 1), k, jnp.int32),
                             dn, (1,),
                             mode=lax.GatherScatterMode.PROMISE_IN_BOUNDS)
                  for k in range(TOPK)]
            for cg in range(DIM // 16):
                accs = [ak[k] * rows_v[p, n * TOPK + k, pl.ds(cg * 16, 16)]
                        for k in range(4)]
                for k in range(4, TOPK):
                    accs[k % 4] = accs[k % 4] + ak[k] * rows_v[
                        p, n * TOPK + k, pl.ds(cg * 16, 16)]
                outb[p][pl.ds(n * DIM + cg * 16, 16)] = (
                    (accs[0] + accs[1]) + (accs[2] + accs[3]))
        _oissue(g, p)

    def _pair(i2, carry):
        _group(i2 * 2, 0)
        _group(i2 * 2 + 1, 1)
        return carry

    lax.fori_loop(0, NG // 2, _pair, 0)
    _owait(NG - 2, 0)
    _owait(NG - 1, 1)


def _msg_sc(idx_t, aj_flat, ai_pad, xp):
    mesh = plsc.VectorSubcoreMesh(core_axis_name="c", subcore_axis_name="s")
    f = functools.partial(
        pl.kernel, _msg_sc_body, mesh=mesh,
        out_type=jax.ShapeDtypeStruct((M * DIM,), jnp.float32),
        scratch_types=[
            pltpu.VMEM((NG * IDXW,), jnp.int32),
            pltpu.VMEM((M,), jnp.float32),
            pltpu.VMEM((CHP,), jnp.float32),
            pltpu.VMEM((2, IDXW, 128), jnp.float32),
            pltpu.VMEM((GPN * DIM,), jnp.float32),
            pltpu.VMEM((GPN * DIM,), jnp.float32),
            pltpu.SemaphoreType.DMA,
            pltpu.SemaphoreType.DMA,
            pltpu.SemaphoreType.DMA,
            pltpu.SemaphoreType.DMA,
        ],
        compiler_params=pltpu.CompilerParams(needs_layout_passes=False),
    )()
    return f(idx_t, aj_flat, ai_pad, xp)


def _stats_body(x_ref, s_ref):
    @pl.when(pl.program_id(0) == 0)
    def _():
        s_ref[...] = jnp.zeros((8, DIM), jnp.float32)

    x = x_ref[...]
    s0 = jnp.sum(x, axis=0, keepdims=True)
    s1 = jnp.sum(x * x, axis=0, keepdims=True)
    upd = jnp.concatenate([s0, s1, jnp.zeros((6, DIM), jnp.float32)], axis=0)
    s_ref[...] = s_ref[...] + upd


def _stats(xflat):
    return pl.pallas_call(
        _stats_body,
        grid=(NBLK,),
        in_specs=[pl.BlockSpec((RP, DIM), lambda i: (i, 0))],
        out_specs=pl.BlockSpec((8, DIM), lambda i: (0, 0)),
        out_shape=jax.ShapeDtypeStruct((8, DIM), jnp.float32),
    )(xflat)


def _bn1_body(x_ref, e_ref, s_ref, g_ref, b_ref, o_ref, s2_ref):
    s = s_ref[...]
    mu = s[0:1, :] * (1.0 / M)
    var = s[1:2, :] * (1.0 / M) - mu * mu
    scale = g_ref[...] * lax.rsqrt(var + EPS)
    shift = b_ref[...] - mu * scale
    o = jnp.maximum(x_ref[...] * scale + shift, 0.0) * e_ref[...]
    o_ref[...] = o

    @pl.when(pl.program_id(0) == 0)
    def _():
        s2_ref[...] = jnp.zeros((8, DIM), jnp.float32)

    s0 = jnp.sum(o, axis=0, keepdims=True)
    s1 = jnp.sum(o * o, axis=0, keepdims=True)
    upd = jnp.concatenate([s0, s1, jnp.zeros((6, DIM), jnp.float32)], axis=0)
    s2_ref[...] = s2_ref[...] + upd


def _bn1(msg, emb, stats1, g1, b1):
    return pl.pallas_call(
        _bn1_body,
        grid=(NBLK,),
        in_specs=[pl.BlockSpec((RP, DIM), lambda i: (i, 0)),
                  pl.BlockSpec((RP, DIM), lambda i: (i % EBLK, 0)),
                  pl.BlockSpec((8, DIM), lambda i: (0, 0)),
                  pl.BlockSpec((1, DIM), lambda i: (0, 0)),
                  pl.BlockSpec((1, DIM), lambda i: (0, 0))],
        out_specs=[pl.BlockSpec((RP, DIM), lambda i: (i, 0)),
                   pl.BlockSpec((8, DIM), lambda i: (0, 0))],
        out_shape=[jax.ShapeDtypeStruct((M, DIM), jnp.float32),
                   jax.ShapeDtypeStruct((8, DIM), jnp.float32)],
    )(msg, emb, stats1, g1, b1)


def _out_body(o_ref, s_ref, g_ref, b_ref, w_ref, ob_ref, y_ref):
    s = s_ref[...]
    mu = s[0:1, :] * (1.0 / M)
    var = s[1:2, :] * (1.0 / M) - mu * mu
    scale = g_ref[...] * lax.rsqrt(var + EPS)
    shift = b_ref[...] - mu * scale
    o = jnp.maximum(o_ref[...] * scale + shift, 0.0)
    y = jnp.sum(o * w_ref[...], axis=1) + ob_ref[0, 0]
    y_ref[...] = y.reshape(1, 1, RP)


def _outk(o, stats2, g2, b2, wrow, ob):
    return pl.pallas_call(
        _out_body,
        grid=(NBLK,),
        in_specs=[pl.BlockSpec((RP, DIM), lambda i: (i, 0)),
                  pl.BlockSpec((8, DIM), lambda i: (0, 0)),
                  pl.BlockSpec((1, DIM), lambda i: (0, 0)),
                  pl.BlockSpec((1, DIM), lambda i: (0, 0)),
                  pl.BlockSpec((1, DIM), lambda i: (0, 0)),
                  pl.BlockSpec((1, 1), lambda i: (0, 0))],
        out_specs=pl.BlockSpec((1, 1, RP), lambda i: (i, 0, 0)),
        out_shape=jax.ShapeDtypeStruct((NBLK, 1, RP), jnp.float32),
    )(o, stats2, g2, b2, wrow, ob)


def kernel(data, org_edge_index, emb, lin_W, lin_b, att_i, att_j, gnn_bias,
           bn1_g, bn1_b, bn2_g, bn2_b, out_W, out_b):
    del org_edge_index, gnn_bias

    embp = jnp.pad(emb, ((0, NPAD - N), (0, 0)))
    embn = _embn(embp)
    topk_idx = _topk(embn)[:N]

    xflat = data.reshape(M, FIN)
    xp, ai3, aj3 = _prep(
        xflat, emb, lin_W, lin_b.reshape(1, DIM),
        att_i[:DIM].reshape(1, DIM), att_i[DIM:].reshape(1, DIM),
        att_j[:DIM].reshape(1, DIM), att_j[DIM:].reshape(1, DIM))
    ai = ai3.reshape(M)
    aj = aj3.reshape(M)

    offs = (jnp.arange(B, dtype=jnp.int32) * N)[:, None, None]
    gidx = (topk_idx[None].astype(jnp.int32) + offs).reshape(-1)
    ai_pad = jnp.pad(ai.reshape(NTILES, CH),
                     ((0, 0), (0, CHP - CH))).reshape(-1)

    msg = _msg_sc(gidx, aj, ai_pad, xp).reshape(M, DIM)

    stats1 = _stats(msg)
    o, stats2 = _bn1(msg, emb, stats1, bn1_g.reshape(1, DIM),
                     bn1_b.reshape(1, DIM))
    y = _outk(o, stats2, bn2_g.reshape(1, DIM), bn2_b.reshape(1, DIM),
              out_W.reshape(1, DIM), out_b.reshape(1, 1))
    return y.reshape(B, N)


# speedup vs baseline: 27.8285x; 1.0063x over previous
import functools

import jax
import jax.numpy as jnp
from jax import lax
from jax.experimental import pallas as pl
from jax.experimental.pallas import tpu as pltpu
from jax.experimental.pallas import tpu_sc as plsc

N = 10000
B = 4
FIN = 10
DIM = 64
TOPK = 16
M = B * N
NPAD = 10112
RB = 128
RP = 1000
NBLK = M // RP
EBLK = N // RP
EPS = 1e-5

NTILES = 32
CH = M // NTILES
CHP = 1256
GPN = 5
NG = CH // GPN
IDXW = GPN * TOPK


def _embn_body(e_ref, o_ref):
    x = e_ref[...]
    nrm = jnp.sqrt(jnp.sum(x * x, axis=1, keepdims=True))
    o_ref[...] = x / (nrm + 1e-12)


def _embn(embp):
    return pl.pallas_call(
        _embn_body,
        grid=(8,),
        in_specs=[pl.BlockSpec((NPAD // 8, DIM), lambda i: (i, 0))],
        out_specs=pl.BlockSpec((NPAD // 8, DIM), lambda i: (i, 0)),
        out_shape=jax.ShapeDtypeStruct((NPAD, DIM), jnp.float32),
    )(embp)


NCH = NPAD // 128


def _topk_body(a_ref, all_ref, idx_ref):
    a = a_ref[...]
    bfull = all_ref[...]
    half = (NCH // 2) * 128
    sim_a = lax.dot_general(a, bfull[:half], (((1,), (1,)), ((), ())),
                            preferred_element_type=jnp.float32)
    sim_b = lax.dot_general(a, bfull[half:], (((1,), (1,)), ((), ())),
                            preferred_element_type=jnp.float32)
    neg = jnp.float32(-3e38)
    big = jnp.int32(2**30)
    lanes = lax.broadcasted_iota(jnp.int32, (RB, 128), 1)
    zed = jnp.zeros((RB, 128), jnp.int32)

    def chunk(c):
        off = c * 128
        if off < half:
            s = sim_a[:, off:off + 128]
        else:
            s = sim_b[:, off - half:off - half + 128]
        if (c + 1) * 128 > N:
            s = jnp.where(lanes < N - off, s, neg)
        return s
    v0 = chunk(0)
    c0 = zed
    v1 = v2 = v3 = jnp.full((RB, 128), neg)
    c1 = c2 = c3 = zed
    for c in range(1, NCH):
        s = chunk(c)
        b0 = s > v0
        b1 = s > v1
        b2 = s > v2
        b3 = s > v3
        v3 = jnp.where(b3, jnp.where(b2, v2, s), v3)
        c3 = jnp.where(b3, jnp.where(b2, c2, c), c3)
        v2 = jnp.where(b2, jnp.where(b1, v1, s), v2)
        c2 = jnp.where(b2, jnp.where(b1, c1, c), c2)
        v1 = jnp.where(b1, jnp.where(b0, v0, s), v1)
        c1 = jnp.where(b1, jnp.where(b0, c0, c), c1)
        v0 = jnp.where(b0, s, v0)
        c0 = jnp.where(b0, c, c0)

    V = jnp.concatenate([v0, v1, v2, v3], axis=1)
    G = jnp.concatenate([c0 * 128 + lanes, c1 * 128 + lanes,
                         c2 * 128 + lanes, c3 * 128 + lanes], axis=1)
    cols = []
    v16 = None
    for k in range(TOPK):
        m = jnp.max(V, axis=1, keepdims=True)
        candi = jnp.where(V >= m, G, big)
        mi = jnp.min(candi, axis=1, keepdims=True)
        cols.append(mi)
        V = jnp.where(candi == mi, neg, V)
        if k == TOPK - 1:
            v16 = m
    idx_ref[...] = jnp.concatenate(cols, axis=1)

    colf_b = half + lax.broadcasted_iota(jnp.int32, (RB, NPAD - half), 1)
    valid_b = colf_b < N
    ge = (jnp.sum(jnp.where(sim_a >= v16, 1.0, 0.0), axis=1, keepdims=True)
          + jnp.sum(jnp.where(valid_b & (sim_b >= v16), 1.0, 0.0), axis=1,
                    keepdims=True))
    nbad = jnp.sum(jnp.where(ge == 16.0, 0.0, 1.0))

    @pl.when(nbad > 0.0)
    def _():
        colf = lax.broadcasted_iota(jnp.int32, (RB, NPAD), 1)
        smat = jnp.concatenate([sim_a, sim_b], axis=1)
        smat = jnp.where(colf < N, smat, neg)
        outc = []
        for _ in range(TOPK):
            m = jnp.max(smat, axis=1, keepdims=True)
            cand = jnp.where(smat >= m, colf, big)
            idx = jnp.min(cand, axis=1, keepdims=True)
            outc.append(idx)
            smat = jnp.where(colf == idx, neg, smat)
        idx_ref[...] = jnp.concatenate(outc, axis=1)


def _topk(embn):
    return pl.pallas_call(
        _topk_body,
        grid=(NPAD // RB,),
        in_specs=[pl.BlockSpec((RB, DIM), lambda i: (i, 0)),
                  pl.BlockSpec((NPAD, DIM), lambda i: (0, 0))],
        out_specs=pl.BlockSpec((RB, TOPK), lambda i: (i, 0)),
        out_shape=jax.ShapeDtypeStruct((NPAD, TOPK), jnp.int32),
        compiler_params=pltpu.CompilerParams(
            vmem_limit_bytes=100 * 1024 * 1024),
    )(embn, embn)


def _prep_body(x_ref, e_ref, w_ref, b_ref, aix_ref, aie_ref, ajx_ref, aje_ref,
               xp_ref, ai_ref, aj_ref):
    x = x_ref[...]
    emb = e_ref[...]
    xp = lax.dot_general(x, w_ref[...], (((1,), (0,)), ((), ())),
                         preferred_element_type=jnp.float32) + b_ref[...]
    xp_ref[...] = jnp.concatenate(
        [xp, jnp.zeros((RP, 128 - DIM), jnp.float32)], axis=1)
    ai = jnp.sum(xp * aix_ref[...], axis=1) + jnp.sum(emb * aie_ref[...], axis=1)
    aj = jnp.sum(xp * ajx_ref[...], axis=1) + jnp.sum(emb * aje_ref[...], axis=1)
    ai_ref[...] = ai.reshape(1, 1, RP)
    aj_ref[...] = aj.reshape(1, 1, RP)


def _prep(xflat, emb, lin_W, lin_b2, aix, aie, ajx, aje):
    return pl.pallas_call(
        _prep_body,
        grid=(NBLK,),
        in_specs=[pl.BlockSpec((RP, FIN), lambda i: (i, 0)),
                  pl.BlockSpec((RP, DIM), lambda i: (i % EBLK, 0)),
                  pl.BlockSpec((FIN, DIM), lambda i: (0, 0)),
                  pl.BlockSpec((1, DIM), lambda i: (0, 0)),
                  pl.BlockSpec((1, DIM), lambda i: (0, 0)),
                  pl.BlockSpec((1, DIM), lambda i: (0, 0)),
                  pl.BlockSpec((1, DIM), lambda i: (0, 0)),
                  pl.BlockSpec((1, DIM), lambda i: (0, 0))],
        out_specs=[pl.BlockSpec((RP, 128), lambda i: (i, 0)),
                   pl.BlockSpec((1, 1, RP), lambda i: (i, 0, 0)),
                   pl.BlockSpec((1, 1, RP), lambda i: (i, 0, 0))],
        out_shape=[jax.ShapeDtypeStruct((M, 128), jnp.float32),
                   jax.ShapeDtypeStruct((NBLK, 1, RP), jnp.float32),
                   jax.ShapeDtypeStruct((NBLK, 1, RP), jnp.float32)],
    )(xflat, emb, lin_W, lin_b2, aix, aie, ajx, aje)


def _msg_sc_body(idx_hbm, aj_hbm, ai_hbm, xp_hbm, out_hbm,
                 idx_v, aj_v, ai_v, rows_v, outb0_v, outb1_v,
                 sg0, sg1, so0, so1):
    c = lax.axis_index("c")
    s = lax.axis_index("s")
    wid = c * 16 + s
    node0 = wid * CH

    pltpu.sync_copy(idx_hbm.at[pl.ds(wid * (NG * IDXW), NG * IDXW)], idx_v)
    pltpu.sync_copy(aj_hbm, aj_v)
    pltpu.sync_copy(ai_hbm.at[pl.ds(wid * CHP, CHP)], ai_v)

    gsem = [sg0, sg1]
    osem = [so0, so1]
    outb = [outb0_v, outb1_v]

    def _gissue(g, p):
        pltpu.async_copy(xp_hbm.at[idx_v.at[pl.ds(g * IDXW, IDXW)]],
                         rows_v.at[p], gsem[p])

    def _gwait(g, p):
        pltpu.make_async_copy(xp_hbm.at[idx_v.at[pl.ds(g * IDXW, IDXW)]],
                              rows_v.at[p], gsem[p]).wait()

    def _oissue(g, p):
        pltpu.async_copy(
            outb[p],
            out_hbm.at[pl.ds((node0 + g * GPN) * DIM, GPN * DIM)], osem[p])

    def _owait(g, p):
        pltpu.make_async_copy(
            outb[p],
            out_hbm.at[pl.ds((node0 + g * GPN) * DIM, GPN * DIM)],
            osem[p]).wait()

    _gissue(0, 0)

    def _group(g, p):
        _gwait(g, p)

        @pl.when(g + 1 < NG)
        def _():
            _gissue(g + 1, 1 - p)

        @pl.when(g >= 2)
        def _():
            _owait(g - 2, p)

        for n in range(GPN):
            nid = g * GPN + n
            idxv = idx_v[pl.ds(g * IDXW + n * TOPK, TOPK)]
            lj = plsc.load_gather(aj_v, [idxv])
            aib = plsc.load_gather(ai_v, [jnp.full((TOPK,), nid, jnp.int32)])
            e = lj + aib
            e = jnp.where(e > 0, e, e * jnp.float32(0.2))
            ex = jnp.exp(e)
            den = jnp.sum(ex)
            alpha = ex / (den + jnp.float32(1e-16))
            dn = lax.GatherDimensionNumbers(
                offset_dims=(), collapsed_slice_dims=(0,),
                start_index_map=(0,))
            ak = [lax.gather(alpha, jnp.full((TOPK, 1), k, jnp.int32),
                             dn, (1,),
                             mode=lax.GatherScatterMode.PROMISE_IN_BOUNDS)
                  for k in range(TOPK)]
            for cg in range(DIM // 16):
                accs = [ak[k] * rows_v[p, n * TOPK + k, pl.ds(cg * 16, 16)]
                        for k in range(4)]
                for k in range(4, TOPK):
                    accs[k % 4] = accs[k % 4] + ak[k] * rows_v[
                        p, n * TOPK + k, pl.ds(cg * 16, 16)]
                outb[p][pl.ds(n * DIM + cg * 16, 16)] = (
                    (accs[0] + accs[1]) + (accs[2] + accs[3]))
        _oissue(g, p)

    def _pair(i2, carry):
        _group(i2 * 2, 0)
        _group(i2 * 2 + 1, 1)
        return carry

    lax.fori_loop(0, NG // 2, _pair, 0)
    _owait(NG - 2, 0)
    _owait(NG - 1, 1)


def _msg_sc(idx_t, aj_flat, ai_pad, xp):
    mesh = plsc.VectorSubcoreMesh(core_axis_name="c", subcore_axis_name="s")
    f = functools.partial(
        pl.kernel, _msg_sc_body, mesh=mesh,
        out_type=jax.ShapeDtypeStruct((M * DIM,), jnp.float32),
        scratch_types=[
            pltpu.VMEM((NG * IDXW,), jnp.int32),
            pltpu.VMEM((M,), jnp.float32),
            pltpu.VMEM((CHP,), jnp.float32),
            pltpu.VMEM((2, IDXW, 128), jnp.float32),
            pltpu.VMEM((GPN * DIM,), jnp.float32),
            pltpu.VMEM((GPN * DIM,), jnp.float32),
            pltpu.SemaphoreType.DMA,
            pltpu.SemaphoreType.DMA,
            pltpu.SemaphoreType.DMA,
            pltpu.SemaphoreType.DMA,
        ],
        compiler_params=pltpu.CompilerParams(needs_layout_passes=False),
    )()
    return f(idx_t, aj_flat, ai_pad, xp)


def _stats_body(x_ref, s_ref):
    @pl.when(pl.program_id(0) == 0)
    def _():
        s_ref[...] = jnp.zeros((8, DIM), jnp.float32)

    x = x_ref[...]
    s0 = jnp.sum(x, axis=0, keepdims=True)
    s1 = jnp.sum(x * x, axis=0, keepdims=True)
    upd = jnp.concatenate([s0, s1, jnp.zeros((6, DIM), jnp.float32)], axis=0)
    s_ref[...] = s_ref[...] + upd


def _stats(xflat):
    return pl.pallas_call(
        _stats_body,
        grid=(NBLK,),
        in_specs=[pl.BlockSpec((RP, DIM), lambda i: (i, 0))],
        out_specs=pl.BlockSpec((8, DIM), lambda i: (0, 0)),
        out_shape=jax.ShapeDtypeStruct((8, DIM), jnp.float32),
    )(xflat)


def _bn1_body(x_ref, e_ref, s_ref, g_ref, b_ref, o_ref, s2_ref):
    s = s_ref[...]
    mu = s[0:1, :] * (1.0 / M)
    var = s[1:2, :] * (1.0 / M) - mu * mu
    scale = g_ref[...] * lax.rsqrt(var + EPS)
    shift = b_ref[...] - mu * scale
    o = jnp.maximum(x_ref[...] * scale + shift, 0.0) * e_ref[...]
    o_ref[...] = o

    @pl.when(pl.program_id(0) == 0)
    def _():
        s2_ref[...] = jnp.zeros((8, DIM), jnp.float32)

    s0 = jnp.sum(o, axis=0, keepdims=True)
    s1 = jnp.sum(o * o, axis=0, keepdims=True)
    upd = jnp.concatenate([s0, s1, jnp.zeros((6, DIM), jnp.float32)], axis=0)
    s2_ref[...] = s2_ref[...] + upd


def _bn1(msg, emb, stats1, g1, b1):
    return pl.pallas_call(
        _bn1_body,
        grid=(NBLK,),
        in_specs=[pl.BlockSpec((RP, DIM), lambda i: (i, 0)),
                  pl.BlockSpec((RP, DIM), lambda i: (i % EBLK, 0)),
                  pl.BlockSpec((8, DIM), lambda i: (0, 0)),
                  pl.BlockSpec((1, DIM), lambda i: (0, 0)),
                  pl.BlockSpec((1, DIM), lambda i: (0, 0))],
        out_specs=[pl.BlockSpec((RP, DIM), lambda i: (i, 0)),
                   pl.BlockSpec((8, DIM), lambda i: (0, 0))],
        out_shape=[jax.ShapeDtypeStruct((M, DIM), jnp.float32),
                   jax.ShapeDtypeStruct((8, DIM), jnp.float32)],
    )(msg, emb, stats1, g1, b1)


def _out_body(o_ref, s_ref, g_ref, b_ref, w_ref, ob_ref, y_ref):
    s = s_ref[...]
    mu = s[0:1, :] * (1.0 / M)
    var = s[1:2, :] * (1.0 / M) - mu * mu
    scale = g_ref[...] * lax.rsqrt(var + EPS)
    shift = b_ref[...] - mu * scale
    o = jnp.maximum(o_ref[...] * scale + shift, 0.0)
    y = jnp.sum(o * w_ref[...], axis=1) + ob_ref[0, 0]
    y_ref[...] = y.reshape(1, 1, RP)


def _outk(o, stats2, g2, b2, wrow, ob):
    return pl.pallas_call(
        _out_body,
        grid=(NBLK,),
        in_specs=[pl.BlockSpec((RP, DIM), lambda i: (i, 0)),
                  pl.BlockSpec((8, DIM), lambda i: (0, 0)),
                  pl.BlockSpec((1, DIM), lambda i: (0, 0)),
                  pl.BlockSpec((1, DIM), lambda i: (0, 0)),
                  pl.BlockSpec((1, DIM), lambda i: (0, 0)),
                  pl.BlockSpec((1, 1), lambda i: (0, 0))],
        out_specs=pl.BlockSpec((1, 1, RP), lambda i: (i, 0, 0)),
        out_shape=jax.ShapeDtypeStruct((NBLK, 1, RP), jnp.float32),
    )(o, stats2, g2, b2, wrow, ob)


def kernel(data, org_edge_index, emb, lin_W, lin_b, att_i, att_j, gnn_bias,
           bn1_g, bn1_b, bn2_g, bn2_b, out_W, out_b):
    del org_edge_index, gnn_bias

    embp = jnp.pad(emb, ((0, NPAD - N), (0, 0)))
    embn = _embn(embp)
    topk_idx = _topk(embn)[:N]

    xflat = data.reshape(M, FIN)
    xp, ai3, aj3 = _prep(
        xflat, emb, lin_W, lin_b.reshape(1, DIM),
        att_i[:DIM].reshape(1, DIM), att_i[DIM:].reshape(1, DIM),
        att_j[:DIM].reshape(1, DIM), att_j[DIM:].reshape(1, DIM))
    ai = ai3.reshape(M)
    aj = aj3.reshape(M)

    offs = (jnp.arange(B, dtype=jnp.int32) * N)[:, None, None]
    gidx = (topk_idx[None].astype(jnp.int32) + offs).reshape(-1)
    ai_pad = jnp.pad(ai.reshape(NTILES, CH),
                     ((0, 0), (0, CHP - CH))).reshape(-1)

    msg = _msg_sc(gidx, aj, ai_pad, xp).reshape(M, DIM)

    stats1 = _stats(msg)
    o, stats2 = _bn1(msg, emb, stats1, bn1_g.reshape(1, DIM),
                     bn1_b.reshape(1, DIM))
    y = _outk(o, stats2, bn2_g.reshape(1, DIM), bn2_b.reshape(1, DIM),
              out_W.reshape(1, DIM), out_b.reshape(1, 1))
    return y.reshape(B, N)


# speedup vs baseline: 28.5532x; 1.0260x over previous
import functools

import jax
import jax.numpy as jnp
from jax import lax
from jax.experimental import pallas as pl
from jax.experimental.pallas import tpu as pltpu
from jax.experimental.pallas import tpu_sc as plsc

N = 10000
B = 4
FIN = 10
DIM = 64
TOPK = 16
M = B * N
NPAD = 10112
RB = 128
RP = 2000
NBLK = M // RP
EBLK = N // RP
EPS = 1e-5

NTILES = 32
CH = M // NTILES
CHP = 1256
GPN = 5
NG = CH // GPN
IDXW = GPN * TOPK


def _embn_body(e_ref, o_ref):
    x = e_ref[...]
    nrm = jnp.sqrt(jnp.sum(x * x, axis=1, keepdims=True))
    o_ref[...] = x / (nrm + 1e-12)


def _embn(embp):
    return pl.pallas_call(
        _embn_body,
        grid=(8,),
        in_specs=[pl.BlockSpec((NPAD // 8, DIM), lambda i: (i, 0))],
        out_specs=pl.BlockSpec((NPAD // 8, DIM), lambda i: (i, 0)),
        out_shape=jax.ShapeDtypeStruct((NPAD, DIM), jnp.float32),
    )(embp)


NCH = NPAD // 128


def _topk_body(a_ref, all_ref, idx_ref):
    a = a_ref[...]
    bfull = all_ref[...]
    half = (NCH // 2) * 128
    sim_a = lax.dot_general(a, bfull[:half], (((1,), (1,)), ((), ())),
                            preferred_element_type=jnp.float32)
    sim_b = lax.dot_general(a, bfull[half:], (((1,), (1,)), ((), ())),
                            preferred_element_type=jnp.float32)
    neg = jnp.float32(-3e38)
    big = jnp.int32(2**30)
    lanes = lax.broadcasted_iota(jnp.int32, (RB, 128), 1)
    zed = jnp.zeros((RB, 128), jnp.int32)

    def chunk(c):
        off = c * 128
        if off < half:
            s = sim_a[:, off:off + 128]
        else:
            s = sim_b[:, off - half:off - half + 128]
        if (c + 1) * 128 > N:
            s = jnp.where(lanes < N - off, s, neg)
        return s
    v0 = chunk(0)
    c0 = zed
    v1 = v2 = v3 = jnp.full((RB, 128), neg)
    c1 = c2 = c3 = zed
    for c in range(1, NCH):
        s = chunk(c)
        b0 = s > v0
        b1 = s > v1
        b2 = s > v2
        b3 = s > v3
        v3 = jnp.where(b3, jnp.where(b2, v2, s), v3)
        c3 = jnp.where(b3, jnp.where(b2, c2, c), c3)
        v2 = jnp.where(b2, jnp.where(b1, v1, s), v2)
        c2 = jnp.where(b2, jnp.where(b1, c1, c), c2)
        v1 = jnp.where(b1, jnp.where(b0, v0, s), v1)
        c1 = jnp.where(b1, jnp.where(b0, c0, c), c1)
        v0 = jnp.where(b0, s, v0)
        c0 = jnp.where(b0, c, c0)

    V = jnp.concatenate([v0, v1, v2, v3], axis=1)
    G = jnp.concatenate([c0 * 128 + lanes, c1 * 128 + lanes,
                         c2 * 128 + lanes, c3 * 128 + lanes], axis=1)
    cols = []
    v16 = None
    for k in range(TOPK):
        m = jnp.max(V, axis=1, keepdims=True)
        candi = jnp.where(V >= m, G, big)
        mi = jnp.min(candi, axis=1, keepdims=True)
        cols.append(mi)
        V = jnp.where(candi == mi, neg, V)
        if k == TOPK - 1:
            v16 = m
    idx_ref[...] = jnp.concatenate(cols, axis=1)

    colf_b = half + lax.broadcasted_iota(jnp.int32, (RB, NPAD - half), 1)
    valid_b = colf_b < N
    ge = (jnp.sum(jnp.where(sim_a >= v16, 1.0, 0.0), axis=1, keepdims=True)
          + jnp.sum(jnp.where(valid_b & (sim_b >= v16), 1.0, 0.0), axis=1,
                    keepdims=True))
    nbad = jnp.sum(jnp.where(ge == 16.0, 0.0, 1.0))

    @pl.when(nbad > 0.0)
    def _():
        colf = lax.broadcasted_iota(jnp.int32, (RB, NPAD), 1)
        smat = jnp.concatenate([sim_a, sim_b], axis=1)
        smat = jnp.where(colf < N, smat, neg)
        outc = []
        for _ in range(TOPK):
            m = jnp.max(smat, axis=1, keepdims=True)
            cand = jnp.where(smat >= m, colf, big)
            idx = jnp.min(cand, axis=1, keepdims=True)
            outc.append(idx)
            smat = jnp.where(colf == idx, neg, smat)
        idx_ref[...] = jnp.concatenate(outc, axis=1)


def _topk(embn):
    return pl.pallas_call(
        _topk_body,
        grid=(NPAD // RB,),
        in_specs=[pl.BlockSpec((RB, DIM), lambda i: (i, 0)),
                  pl.BlockSpec((NPAD, DIM), lambda i: (0, 0))],
        out_specs=pl.BlockSpec((RB, TOPK), lambda i: (i, 0)),
        out_shape=jax.ShapeDtypeStruct((NPAD, TOPK), jnp.int32),
        compiler_params=pltpu.CompilerParams(
            vmem_limit_bytes=100 * 1024 * 1024),
    )(embn, embn)


def _prep_body(x_ref, e_ref, w_ref, b_ref, aix_ref, aie_ref, ajx_ref, aje_ref,
               xp_ref, ai_ref, aj_ref):
    x = x_ref[...]
    emb = e_ref[...]
    xp = lax.dot_general(x, w_ref[...], (((1,), (0,)), ((), ())),
                         preferred_element_type=jnp.float32) + b_ref[...]
    xp_ref[...] = jnp.concatenate(
        [xp, jnp.zeros((RP, 128 - DIM), jnp.float32)], axis=1)
    ai = jnp.sum(xp * aix_ref[...], axis=1) + jnp.sum(emb * aie_ref[...], axis=1)
    aj = jnp.sum(xp * ajx_ref[...], axis=1) + jnp.sum(emb * aje_ref[...], axis=1)
    ai_ref[...] = ai.reshape(1, 1, RP)
    aj_ref[...] = aj.reshape(1, 1, RP)


def _prep(xflat, emb, lin_W, lin_b2, aix, aie, ajx, aje):
    return pl.pallas_call(
        _prep_body,
        grid=(NBLK,),
        in_specs=[pl.BlockSpec((RP, FIN), lambda i: (i, 0)),
                  pl.BlockSpec((RP, DIM), lambda i: (i % EBLK, 0)),
                  pl.BlockSpec((FIN, DIM), lambda i: (0, 0)),
                  pl.BlockSpec((1, DIM), lambda i: (0, 0)),
                  pl.BlockSpec((1, DIM), lambda i: (0, 0)),
                  pl.BlockSpec((1, DIM), lambda i: (0, 0)),
                  pl.BlockSpec((1, DIM), lambda i: (0, 0)),
                  pl.BlockSpec((1, DIM), lambda i: (0, 0))],
        out_specs=[pl.BlockSpec((RP, 128), lambda i: (i, 0)),
                   pl.BlockSpec((1, 1, RP), lambda i: (i, 0, 0)),
                   pl.BlockSpec((1, 1, RP), lambda i: (i, 0, 0))],
        out_shape=[jax.ShapeDtypeStruct((M, 128), jnp.float32),
                   jax.ShapeDtypeStruct((NBLK, 1, RP), jnp.float32),
                   jax.ShapeDtypeStruct((NBLK, 1, RP), jnp.float32)],
    )(xflat, emb, lin_W, lin_b2, aix, aie, ajx, aje)


def _msg_sc_body(idx_hbm, aj_hbm, ai_hbm, xp_hbm, out_hbm,
                 idx_v, aj_v, ai_v, rows_v, outb0_v, outb1_v,
                 sg0, sg1, so0, so1):
    c = lax.axis_index("c")
    s = lax.axis_index("s")
    wid = c * 16 + s
    node0 = wid * CH

    pltpu.sync_copy(idx_hbm.at[pl.ds(wid * (NG * IDXW), NG * IDXW)], idx_v)
    pltpu.sync_copy(aj_hbm, aj_v)
    pltpu.sync_copy(ai_hbm.at[pl.ds(wid * CHP, CHP)], ai_v)

    gsem = [sg0, sg1]
    osem = [so0, so1]
    outb = [outb0_v, outb1_v]

    def _gissue(g, p):
        pltpu.async_copy(xp_hbm.at[idx_v.at[pl.ds(g * IDXW, IDXW)]],
                         rows_v.at[p], gsem[p])

    def _gwait(g, p):
        pltpu.make_async_copy(xp_hbm.at[idx_v.at[pl.ds(g * IDXW, IDXW)]],
                              rows_v.at[p], gsem[p]).wait()

    def _oissue(g, p):
        pltpu.async_copy(
            outb[p],
            out_hbm.at[pl.ds((node0 + g * GPN) * DIM, GPN * DIM)], osem[p])

    def _owait(g, p):
        pltpu.make_async_copy(
            outb[p],
            out_hbm.at[pl.ds((node0 + g * GPN) * DIM, GPN * DIM)],
            osem[p]).wait()

    _gissue(0, 0)

    def _group(g, p):
        _gwait(g, p)

        @pl.when(g + 1 < NG)
        def _():
            _gissue(g + 1, 1 - p)

        @pl.when(g >= 2)
        def _():
            _owait(g - 2, p)

        for n in range(GPN):
            nid = g * GPN + n
            idxv = idx_v[pl.ds(g * IDXW + n * TOPK, TOPK)]
            lj = plsc.load_gather(aj_v, [idxv])
            aib = plsc.load_gather(ai_v, [jnp.full((TOPK,), nid, jnp.int32)])
            e = lj + aib
            e = jnp.where(e > 0, e, e * jnp.float32(0.2))
            ex = jnp.exp(e)
            den = jnp.sum(ex)
            alpha = ex / (den + jnp.float32(1e-16))
            dn = lax.GatherDimensionNumbers(
                offset_dims=(), collapsed_slice_dims=(0,),
                start_index_map=(0,))
            ak = [lax.gather(alpha, jnp.full((TOPK, 1), k, jnp.int32),
                             dn, (1,),
                             mode=lax.GatherScatterMode.PROMISE_IN_BOUNDS)
                  for k in range(TOPK)]
            for cg in range(DIM // 16):
                accs = [ak[k] * rows_v[p, n * TOPK + k, pl.ds(cg * 16, 16)]
                        for k in range(4)]
                for k in range(4, TOPK):
                    accs[k % 4] = accs[k % 4] + ak[k] * rows_v[
                        p, n * TOPK + k, pl.ds(cg * 16, 16)]
                outb[p][pl.ds(n * DIM + cg * 16, 16)] = (
                    (accs[0] + accs[1]) + (accs[2] + accs[3]))
        _oissue(g, p)

    def _pair(i2, carry):
        _group(i2 * 2, 0)
        _group(i2 * 2 + 1, 1)
        return carry

    lax.fori_loop(0, NG // 2, _pair, 0)
    _owait(NG - 2, 0)
    _owait(NG - 1, 1)


def _msg_sc(idx_t, aj_flat, ai_pad, xp):
    mesh = plsc.VectorSubcoreMesh(core_axis_name="c", subcore_axis_name="s")
    f = functools.partial(
        pl.kernel, _msg_sc_body, mesh=mesh,
        out_type=jax.ShapeDtypeStruct((M * DIM,), jnp.float32),
        scratch_types=[
            pltpu.VMEM((NG * IDXW,), jnp.int32),
            pltpu.VMEM((M,), jnp.float32),
            pltpu.VMEM((CHP,), jnp.float32),
            pltpu.VMEM((2, IDXW, 128), jnp.float32),
            pltpu.VMEM((GPN * DIM,), jnp.float32),
            pltpu.VMEM((GPN * DIM,), jnp.float32),
            pltpu.SemaphoreType.DMA,
            pltpu.SemaphoreType.DMA,
            pltpu.SemaphoreType.DMA,
            pltpu.SemaphoreType.DMA,
        ],
        compiler_params=pltpu.CompilerParams(needs_layout_passes=False),
    )()
    return f(idx_t, aj_flat, ai_pad, xp)


def _stats_body(x_ref, s_ref):
    @pl.when(pl.program_id(0) == 0)
    def _():
        s_ref[...] = jnp.zeros((8, DIM), jnp.float32)

    x = x_ref[...]
    s0 = jnp.sum(x, axis=0, keepdims=True)
    s1 = jnp.sum(x * x, axis=0, keepdims=True)
    upd = jnp.concatenate([s0, s1, jnp.zeros((6, DIM), jnp.float32)], axis=0)
    s_ref[...] = s_ref[...] + upd


def _stats(xflat):
    return pl.pallas_call(
        _stats_body,
        grid=(NBLK,),
        in_specs=[pl.BlockSpec((RP, DIM), lambda i: (i, 0))],
        out_specs=pl.BlockSpec((8, DIM), lambda i: (0, 0)),
        out_shape=jax.ShapeDtypeStruct((8, DIM), jnp.float32),
    )(xflat)


def _bn1_body(x_ref, e_ref, s_ref, g_ref, b_ref, o_ref, s2_ref):
    s = s_ref[...]
    mu = s[0:1, :] * (1.0 / M)
    var = s[1:2, :] * (1.0 / M) - mu * mu
    scale = g_ref[...] * lax.rsqrt(var + EPS)
    shift = b_ref[...] - mu * scale
    o = jnp.maximum(x_ref[...] * scale + shift, 0.0) * e_ref[...]
    o_ref[...] = o

    @pl.when(pl.program_id(0) == 0)
    def _():
        s2_ref[...] = jnp.zeros((8, DIM), jnp.float32)

    s0 = jnp.sum(o, axis=0, keepdims=True)
    s1 = jnp.sum(o * o, axis=0, keepdims=True)
    upd = jnp.concatenate([s0, s1, jnp.zeros((6, DIM), jnp.float32)], axis=0)
    s2_ref[...] = s2_ref[...] + upd


def _bn1(msg, emb, stats1, g1, b1):
    return pl.pallas_call(
        _bn1_body,
        grid=(NBLK,),
        in_specs=[pl.BlockSpec((RP, DIM), lambda i: (i, 0)),
                  pl.BlockSpec((RP, DIM), lambda i: (i % EBLK, 0)),
                  pl.BlockSpec((8, DIM), lambda i: (0, 0)),
                  pl.BlockSpec((1, DIM), lambda i: (0, 0)),
                  pl.BlockSpec((1, DIM), lambda i: (0, 0))],
        out_specs=[pl.BlockSpec((RP, DIM), lambda i: (i, 0)),
                   pl.BlockSpec((8, DIM), lambda i: (0, 0))],
        out_shape=[jax.ShapeDtypeStruct((M, DIM), jnp.float32),
                   jax.ShapeDtypeStruct((8, DIM), jnp.float32)],
    )(msg, emb, stats1, g1, b1)


def _out_body(o_ref, s_ref, g_ref, b_ref, w_ref, ob_ref, y_ref):
    s = s_ref[...]
    mu = s[0:1, :] * (1.0 / M)
    var = s[1:2, :] * (1.0 / M) - mu * mu
    scale = g_ref[...] * lax.rsqrt(var + EPS)
    shift = b_ref[...] - mu * scale
    o = jnp.maximum(o_ref[...] * scale + shift, 0.0)
    y = jnp.sum(o * w_ref[...], axis=1) + ob_ref[0, 0]
    y_ref[...] = y.reshape(1, 1, RP)


def _outk(o, stats2, g2, b2, wrow, ob):
    return pl.pallas_call(
        _out_body,
        grid=(NBLK,),
        in_specs=[pl.BlockSpec((RP, DIM), lambda i: (i, 0)),
                  pl.BlockSpec((8, DIM), lambda i: (0, 0)),
                  pl.BlockSpec((1, DIM), lambda i: (0, 0)),
                  pl.BlockSpec((1, DIM), lambda i: (0, 0)),
                  pl.BlockSpec((1, DIM), lambda i: (0, 0)),
                  pl.BlockSpec((1, 1), lambda i: (0, 0))],
        out_specs=pl.BlockSpec((1, 1, RP), lambda i: (i, 0, 0)),
        out_shape=jax.ShapeDtypeStruct((NBLK, 1, RP), jnp.float32),
    )(o, stats2, g2, b2, wrow, ob)


def kernel(data, org_edge_index, emb, lin_W, lin_b, att_i, att_j, gnn_bias,
           bn1_g, bn1_b, bn2_g, bn2_b, out_W, out_b):
    del org_edge_index, gnn_bias

    embp = jnp.pad(emb, ((0, NPAD - N), (0, 0)))
    embn = _embn(embp)
    topk_idx = _topk(embn)[:N]

    xflat = data.reshape(M, FIN)
    xp, ai3, aj3 = _prep(
        xflat, emb, lin_W, lin_b.reshape(1, DIM),
        att_i[:DIM].reshape(1, DIM), att_i[DIM:].reshape(1, DIM),
        att_j[:DIM].reshape(1, DIM), att_j[DIM:].reshape(1, DIM))
    ai = ai3.reshape(M)
    aj = aj3.reshape(M)

    offs = (jnp.arange(B, dtype=jnp.int32) * N)[:, None, None]
    gidx = (topk_idx[None].astype(jnp.int32) + offs).reshape(-1)
    ai_pad = jnp.pad(ai.reshape(NTILES, CH),
                     ((0, 0), (0, CHP - CH))).reshape(-1)

    msg = _msg_sc(gidx, aj, ai_pad, xp).reshape(M, DIM)

    stats1 = _stats(msg)
    o, stats2 = _bn1(msg, emb, stats1, bn1_g.reshape(1, DIM),
                     bn1_b.reshape(1, DIM))
    y = _outk(o, stats2, bn2_g.reshape(1, DIM), bn2_b.reshape(1, DIM),
              out_W.reshape(1, DIM), out_b.reshape(1, 1))
    return y.reshape(B, N)


# speedup vs baseline: 28.8945x; 1.0120x over previous
import functools

import jax
import jax.numpy as jnp
from jax import lax
from jax.experimental import pallas as pl
from jax.experimental.pallas import tpu as pltpu
from jax.experimental.pallas import tpu_sc as plsc

N = 10000
B = 4
FIN = 10
DIM = 64
TOPK = 16
M = B * N
NPAD = 10112
RB = 128
RP = 2000
NBLK = M // RP
EBLK = N // RP
EPS = 1e-5

NTILES = 32
CH = M // NTILES
CHP = 1256
GPN = 5
NG = CH // GPN
IDXW = GPN * TOPK


def _embn_body(e_ref, o_ref):
    x = e_ref[...]
    nrm = jnp.sqrt(jnp.sum(x * x, axis=1, keepdims=True))
    o_ref[...] = x / (nrm + 1e-12)


def _embn(embp):
    return pl.pallas_call(
        _embn_body,
        grid=(8,),
        in_specs=[pl.BlockSpec((NPAD // 8, DIM), lambda i: (i, 0))],
        out_specs=pl.BlockSpec((NPAD // 8, DIM), lambda i: (i, 0)),
        out_shape=jax.ShapeDtypeStruct((NPAD, DIM), jnp.float32),
    )(embp)


NCH = NPAD // 128


def _topk_body(a_ref, all_ref, idx_ref):
    a = a_ref[...]
    bfull = all_ref[...]
    half = (NCH // 2) * 128
    sim_a = lax.dot_general(a, bfull[:half], (((1,), (1,)), ((), ())),
                            preferred_element_type=jnp.float32)
    sim_b = lax.dot_general(a, bfull[half:], (((1,), (1,)), ((), ())),
                            preferred_element_type=jnp.float32)
    neg = jnp.float32(-3e38)
    big = jnp.int32(2**30)
    lanes = lax.broadcasted_iota(jnp.int32, (RB, 128), 1)
    zed = jnp.zeros((RB, 128), jnp.int32)

    def chunk(c):
        off = c * 128
        if off < half:
            s = sim_a[:, off:off + 128]
        else:
            s = sim_b[:, off - half:off - half + 128]
        if (c + 1) * 128 > N:
            s = jnp.where(lanes < N - off, s, neg)
        return s
    v0 = chunk(0)
    c0 = zed
    v1 = v2 = v3 = jnp.full((RB, 128), neg)
    c1 = c2 = c3 = zed
    for c in range(1, NCH):
        s = chunk(c)
        b0 = s > v0
        b1 = s > v1
        b2 = s > v2
        b3 = s > v3
        v3 = jnp.where(b3, jnp.where(b2, v2, s), v3)
        c3 = jnp.where(b3, jnp.where(b2, c2, c), c3)
        v2 = jnp.where(b2, jnp.where(b1, v1, s), v2)
        c2 = jnp.where(b2, jnp.where(b1, c1, c), c2)
        v1 = jnp.where(b1, jnp.where(b0, v0, s), v1)
        c1 = jnp.where(b1, jnp.where(b0, c0, c), c1)
        v0 = jnp.where(b0, s, v0)
        c0 = jnp.where(b0, c, c0)

    V = jnp.concatenate([v0, v1, v2, v3], axis=1)
    G = jnp.concatenate([c0 * 128 + lanes, c1 * 128 + lanes,
                         c2 * 128 + lanes, c3 * 128 + lanes], axis=1)
    cols = []
    v16 = None
    for k in range(TOPK):
        m = jnp.max(V, axis=1, keepdims=True)
        candi = jnp.where(V >= m, G, big)
        mi = jnp.min(candi, axis=1, keepdims=True)
        cols.append(mi)
        V = jnp.where(candi == mi, neg, V)
        if k == TOPK - 1:
            v16 = m
    idx_ref[...] = jnp.concatenate(cols, axis=1)

    colf_b = half + lax.broadcasted_iota(jnp.int32, (RB, NPAD - half), 1)
    valid_b = colf_b < N
    ge = (jnp.sum(jnp.where(sim_a >= v16, 1.0, 0.0), axis=1, keepdims=True)
          + jnp.sum(jnp.where(valid_b & (sim_b >= v16), 1.0, 0.0), axis=1,
                    keepdims=True))
    nbad = jnp.sum(jnp.where(ge == 16.0, 0.0, 1.0))

    @pl.when(nbad > 0.0)
    def _():
        colf = lax.broadcasted_iota(jnp.int32, (RB, NPAD), 1)
        smat = jnp.concatenate([sim_a, sim_b], axis=1)
        smat = jnp.where(colf < N, smat, neg)
        outc = []
        for _ in range(TOPK):
            m = jnp.max(smat, axis=1, keepdims=True)
            cand = jnp.where(smat >= m, colf, big)
            idx = jnp.min(cand, axis=1, keepdims=True)
            outc.append(idx)
            smat = jnp.where(colf == idx, neg, smat)
        idx_ref[...] = jnp.concatenate(outc, axis=1)


def _topk(embn):
    return pl.pallas_call(
        _topk_body,
        grid=(NPAD // RB,),
        in_specs=[pl.BlockSpec((RB, DIM), lambda i: (i, 0)),
                  pl.BlockSpec((NPAD, DIM), lambda i: (0, 0))],
        out_specs=pl.BlockSpec((RB, TOPK), lambda i: (i, 0)),
        out_shape=jax.ShapeDtypeStruct((NPAD, TOPK), jnp.int32),
        compiler_params=pltpu.CompilerParams(
            vmem_limit_bytes=100 * 1024 * 1024),
    )(embn, embn)


def _prep_body(x_ref, e_ref, w_ref, b_ref, aix_ref, aie_ref, ajx_ref, aje_ref,
               xp_ref, ai_ref, aj_ref):
    x = x_ref[...]
    emb = e_ref[...]
    xp = lax.dot_general(x, w_ref[...], (((1,), (0,)), ((), ())),
                         preferred_element_type=jnp.float32) + b_ref[...]
    xp_ref[...] = jnp.concatenate(
        [xp, jnp.zeros((RP, 128 - DIM), jnp.float32)], axis=1)
    ai = jnp.sum(xp * aix_ref[...], axis=1) + jnp.sum(emb * aie_ref[...], axis=1)
    aj = jnp.sum(xp * ajx_ref[...], axis=1) + jnp.sum(emb * aje_ref[...], axis=1)
    ai_ref[...] = ai.reshape(1, 1, RP)
    aj_ref[...] = aj.reshape(1, 1, RP)


def _prep(xflat, emb, lin_W, lin_b2, aix, aie, ajx, aje):
    return pl.pallas_call(
        _prep_body,
        grid=(NBLK,),
        in_specs=[pl.BlockSpec((RP, FIN), lambda i: (i, 0)),
                  pl.BlockSpec((RP, DIM), lambda i: (i % EBLK, 0)),
                  pl.BlockSpec((FIN, DIM), lambda i: (0, 0)),
                  pl.BlockSpec((1, DIM), lambda i: (0, 0)),
                  pl.BlockSpec((1, DIM), lambda i: (0, 0)),
                  pl.BlockSpec((1, DIM), lambda i: (0, 0)),
                  pl.BlockSpec((1, DIM), lambda i: (0, 0)),
                  pl.BlockSpec((1, DIM), lambda i: (0, 0))],
        out_specs=[pl.BlockSpec((RP, 128), lambda i: (i, 0)),
                   pl.BlockSpec((1, 1, RP), lambda i: (i, 0, 0)),
                   pl.BlockSpec((1, 1, RP), lambda i: (i, 0, 0))],
        out_shape=[jax.ShapeDtypeStruct((M, 128), jnp.float32),
                   jax.ShapeDtypeStruct((NBLK, 1, RP), jnp.float32),
                   jax.ShapeDtypeStruct((NBLK, 1, RP), jnp.float32)],
    )(xflat, emb, lin_W, lin_b2, aix, aie, ajx, aje)


def _msg_sc_body(idx_hbm, aj_hbm, ai_hbm, xp_hbm, out_hbm, st_hbm,
                 idx_v, aj_v, ai_v, rows_v, outb0_v, outb1_v, st_v,
                 sg0, sg1, so0, so1):
    c = lax.axis_index("c")
    s = lax.axis_index("s")
    wid = c * 16 + s
    node0 = wid * CH

    pltpu.sync_copy(idx_hbm.at[pl.ds(wid * (NG * IDXW), NG * IDXW)], idx_v)
    pltpu.sync_copy(aj_hbm, aj_v)
    pltpu.sync_copy(ai_hbm.at[pl.ds(wid * CHP, CHP)], ai_v)

    gsem = [sg0, sg1]
    osem = [so0, so1]
    outb = [outb0_v, outb1_v]

    def _gissue(g, p):
        pltpu.async_copy(xp_hbm.at[idx_v.at[pl.ds(g * IDXW, IDXW)]],
                         rows_v.at[p], gsem[p])

    def _gwait(g, p):
        pltpu.make_async_copy(xp_hbm.at[idx_v.at[pl.ds(g * IDXW, IDXW)]],
                              rows_v.at[p], gsem[p]).wait()

    def _oissue(g, p):
        pltpu.async_copy(
            outb[p],
            out_hbm.at[pl.ds((node0 + g * GPN) * DIM, GPN * DIM)], osem[p])

    def _owait(g, p):
        pltpu.make_async_copy(
            outb[p],
            out_hbm.at[pl.ds((node0 + g * GPN) * DIM, GPN * DIM)],
            osem[p]).wait()

    _gissue(0, 0)

    def _group(g, p, st):
        _gwait(g, p)

        @pl.when(g + 1 < NG)
        def _():
            _gissue(g + 1, 1 - p)

        @pl.when(g >= 2)
        def _():
            _owait(g - 2, p)

        for n in range(GPN):
            nid = g * GPN + n
            idxv = idx_v[pl.ds(g * IDXW + n * TOPK, TOPK)]
            lj = plsc.load_gather(aj_v, [idxv])
            aib = plsc.load_gather(ai_v, [jnp.full((TOPK,), nid, jnp.int32)])
            e = lj + aib
            e = jnp.where(e > 0, e, e * jnp.float32(0.2))
            ex = jnp.exp(e)
            den = jnp.sum(ex)
            alpha = ex / (den + jnp.float32(1e-16))
            dn = lax.GatherDimensionNumbers(
                offset_dims=(), collapsed_slice_dims=(0,),
                start_index_map=(0,))
            ak = [lax.gather(alpha, jnp.full((TOPK, 1), k, jnp.int32),
                             dn, (1,),
                             mode=lax.GatherScatterMode.PROMISE_IN_BOUNDS)
                  for k in range(TOPK)]
            for cg in range(DIM // 16):
                accs = [ak[k] * rows_v[p, n * TOPK + k, pl.ds(cg * 16, 16)]
                        for k in range(4)]
                for k in range(4, TOPK):
                    accs[k % 4] = accs[k % 4] + ak[k] * rows_v[
                        p, n * TOPK + k, pl.ds(cg * 16, 16)]
                acc = (accs[0] + accs[1]) + (accs[2] + accs[3])
                outb[p][pl.ds(n * DIM + cg * 16, 16)] = acc
                st[cg] = st[cg] + acc
                st[4 + cg] = st[4 + cg] + acc * acc
        _oissue(g, p)
        return st

    def _pair(i2, st):
        st = _group(i2 * 2, 0, list(st))
        st = _group(i2 * 2 + 1, 1, st)
        return tuple(st)

    zero16 = jnp.zeros((16,), jnp.float32)
    st = lax.fori_loop(0, NG // 2, _pair, (zero16,) * 8)
    _owait(NG - 2, 0)
    _owait(NG - 1, 1)
    for j in range(8):
        st_v[pl.ds(j * 16, 16)] = st[j]
    pltpu.sync_copy(st_v, st_hbm.at[pl.ds(wid * 128, 128)])


def _msg_sc(idx_t, aj_flat, ai_pad, xp):
    mesh = plsc.VectorSubcoreMesh(core_axis_name="c", subcore_axis_name="s")
    f = functools.partial(
        pl.kernel, _msg_sc_body, mesh=mesh,
        out_type=[jax.ShapeDtypeStruct((M * DIM,), jnp.float32),
                  jax.ShapeDtypeStruct((NTILES * 128,), jnp.float32)],
        scratch_types=[
            pltpu.VMEM((NG * IDXW,), jnp.int32),
            pltpu.VMEM((M,), jnp.float32),
            pltpu.VMEM((CHP,), jnp.float32),
            pltpu.VMEM((2, IDXW, 128), jnp.float32),
            pltpu.VMEM((GPN * DIM,), jnp.float32),
            pltpu.VMEM((GPN * DIM,), jnp.float32),
            pltpu.VMEM((128,), jnp.float32),
            pltpu.SemaphoreType.DMA,
            pltpu.SemaphoreType.DMA,
            pltpu.SemaphoreType.DMA,
            pltpu.SemaphoreType.DMA,
        ],
        compiler_params=pltpu.CompilerParams(needs_layout_passes=False),
    )()
    return f(idx_t, aj_flat, ai_pad, xp)


def _stats_body(x_ref, s_ref):
    @pl.when(pl.program_id(0) == 0)
    def _():
        s_ref[...] = jnp.zeros((8, DIM), jnp.float32)

    x = x_ref[...]
    s0 = jnp.sum(x, axis=0, keepdims=True)
    s1 = jnp.sum(x * x, axis=0, keepdims=True)
    upd = jnp.concatenate([s0, s1, jnp.zeros((6, DIM), jnp.float32)], axis=0)
    s_ref[...] = s_ref[...] + upd


def _stats(xflat):
    return pl.pallas_call(
        _stats_body,
        grid=(NBLK,),
        in_specs=[pl.BlockSpec((RP, DIM), lambda i: (i, 0))],
        out_specs=pl.BlockSpec((8, DIM), lambda i: (0, 0)),
        out_shape=jax.ShapeDtypeStruct((8, DIM), jnp.float32),
    )(xflat)


def _bn1_body(x_ref, e_ref, s_ref, g_ref, b_ref, o_ref, s2_ref):
    s = s_ref[...]
    mu = jnp.sum(s[:, :DIM], axis=0, keepdims=True) * (1.0 / M)
    var = jnp.sum(s[:, DIM:], axis=0, keepdims=True) * (1.0 / M) - mu * mu
    scale = g_ref[...] * lax.rsqrt(var + EPS)
    shift = b_ref[...] - mu * scale
    o = jnp.maximum(x_ref[...] * scale + shift, 0.0) * e_ref[...]
    o_ref[...] = o

    @pl.when(pl.program_id(0) == 0)
    def _():
        s2_ref[...] = jnp.zeros((8, DIM), jnp.float32)

    s0 = jnp.sum(o, axis=0, keepdims=True)
    s1 = jnp.sum(o * o, axis=0, keepdims=True)
    upd = jnp.concatenate([s0, s1, jnp.zeros((6, DIM), jnp.float32)], axis=0)
    s2_ref[...] = s2_ref[...] + upd


def _bn1(msg, emb, stats1, g1, b1):
    return pl.pallas_call(
        _bn1_body,
        grid=(NBLK,),
        in_specs=[pl.BlockSpec((RP, DIM), lambda i: (i, 0)),
                  pl.BlockSpec((RP, DIM), lambda i: (i % EBLK, 0)),
                  pl.BlockSpec((NTILES, 128), lambda i: (0, 0)),
                  pl.BlockSpec((1, DIM), lambda i: (0, 0)),
                  pl.BlockSpec((1, DIM), lambda i: (0, 0))],
        out_specs=[pl.BlockSpec((RP, DIM), lambda i: (i, 0)),
                   pl.BlockSpec((8, DIM), lambda i: (0, 0))],
        out_shape=[jax.ShapeDtypeStruct((M, DIM), jnp.float32),
                   jax.ShapeDtypeStruct((8, DIM), jnp.float32)],
    )(msg, emb, stats1, g1, b1)


def _out_body(o_ref, s_ref, g_ref, b_ref, w_ref, ob_ref, y_ref):
    s = s_ref[...]
    mu = s[0:1, :] * (1.0 / M)
    var = s[1:2, :] * (1.0 / M) - mu * mu
    scale = g_ref[...] * lax.rsqrt(var + EPS)
    shift = b_ref[...] - mu * scale
    o = jnp.maximum(o_ref[...] * scale + shift, 0.0)
    y = jnp.sum(o * w_ref[...], axis=1) + ob_ref[0, 0]
    y_ref[...] = y.reshape(1, 1, RP)


def _outk(o, stats2, g2, b2, wrow, ob):
    return pl.pallas_call(
        _out_body,
        grid=(NBLK,),
        in_specs=[pl.BlockSpec((RP, DIM), lambda i: (i, 0)),
                  pl.BlockSpec((8, DIM), lambda i: (0, 0)),
                  pl.BlockSpec((1, DIM), lambda i: (0, 0)),
                  pl.BlockSpec((1, DIM), lambda i: (0, 0)),
                  pl.BlockSpec((1, DIM), lambda i: (0, 0)),
                  pl.BlockSpec((1, 1), lambda i: (0, 0))],
        out_specs=pl.BlockSpec((1, 1, RP), lambda i: (i, 0, 0)),
        out_shape=jax.ShapeDtypeStruct((NBLK, 1, RP), jnp.float32),
    )(o, stats2, g2, b2, wrow, ob)


def kernel(data, org_edge_index, emb, lin_W, lin_b, att_i, att_j, gnn_bias,
           bn1_g, bn1_b, bn2_g, bn2_b, out_W, out_b):
    del org_edge_index, gnn_bias

    embp = jnp.pad(emb, ((0, NPAD - N), (0, 0)))
    embn = _embn(embp)
    topk_idx = _topk(embn)[:N]

    xflat = data.reshape(M, FIN)
    xp, ai3, aj3 = _prep(
        xflat, emb, lin_W, lin_b.reshape(1, DIM),
        att_i[:DIM].reshape(1, DIM), att_i[DIM:].reshape(1, DIM),
        att_j[:DIM].reshape(1, DIM), att_j[DIM:].reshape(1, DIM))
    ai = ai3.reshape(M)
    aj = aj3.reshape(M)

    offs = (jnp.arange(B, dtype=jnp.int32) * N)[:, None, None]
    gidx = (topk_idx[None].astype(jnp.int32) + offs).reshape(-1)
    ai_pad = jnp.pad(ai.reshape(NTILES, CH),
                     ((0, 0), (0, CHP - CH))).reshape(-1)

    msg_flat, st = _msg_sc(gidx, aj, ai_pad, xp)
    msg = msg_flat.reshape(M, DIM)

    o, stats2 = _bn1(msg, emb, st.reshape(NTILES, 128),
                     bn1_g.reshape(1, DIM), bn1_b.reshape(1, DIM))
    y = _outk(o, stats2, bn2_g.reshape(1, DIM), bn2_b.reshape(1, DIM),
              out_W.reshape(1, DIM), out_b.reshape(1, 1))
    return y.reshape(B, N)
